# Initial kernel scaffold; baseline (speedup 1.0000x reference)
import jax
import jax.numpy as jnp
from jax.experimental import pallas as pl


def kernel(x, edge_index, batch, W1, b1, W2, b2, Wp, bp, Wc, bc):
    raise NotImplementedError("write your pallas kernel here")


# speedup vs baseline: 1.0297x; 1.0297x over previous
import functools

import jax
import jax.numpy as jnp
from jax import lax
from jax.experimental import pallas as pl
from jax.experimental.pallas import tpu as pltpu

_N = 50000
_E = 800000
_B = 64
_SEQ = 128
_HID = 256
_PROJ = 512
_NCLS = 1000

_RB = 1024
_NP = ((_N + _RB - 1) // _RB) * _RB
_G = _NP // _RB
_MD = 272

_F32 = jnp.float32


def _mlp1_body(x_ref, w_ref, b_ref, o_ref):
    h = jnp.dot(x_ref[...], w_ref[...], preferred_element_type=_F32)
    o_ref[...] = jnp.maximum(h + b_ref[...], 0.0)


def _mlp2_pool_body(h_ref, md_ref, bt_ref, w_ref, b_ref, sums_ref, cnt_ref):
    i = pl.program_id(0)
    msg = md_ref[:, :_HID]
    deg = md_ref[:, _HID:_HID + 1]
    m = msg / jnp.maximum(deg, 1.0)
    Hb = jnp.dot(h_ref[...] + m, w_ref[...], preferred_element_type=_F32)
    Hb = jnp.maximum(Hb + b_ref[...], 0.0)
    bt = bt_ref[0]
    seg = lax.broadcasted_iota(jnp.int32, (_B, _RB), 0)
    mask = (seg == bt).astype(_F32)
    psum = jnp.dot(mask, Hb, preferred_element_type=_F32)
    pcnt = jnp.sum(mask, axis=1, keepdims=True)

    @pl.when(i == 0)
    def _init():
        sums_ref[...] = psum
        cnt_ref[...] = jnp.broadcast_to(pcnt, (_B, 128))

    @pl.when(i > 0)
    def _acc():
        sums_ref[...] += psum
        cnt_ref[...] += jnp.broadcast_to(pcnt, (_B, 128))


def _head_body(sums_ref, cnt_ref, wp_ref, bp_ref, wc_ref, bc_ref, o_ref):
    cnt = cnt_ref[:, 0:1]
    hag = sums_ref[...] / jnp.maximum(cnt, 1.0)
    t = jnp.dot(hag, wp_ref[...], preferred_element_type=_F32) + bp_ref[...]
    o_ref[...] = jnp.dot(t, wc_ref[...], preferred_element_type=_F32) + bc_ref[...]


def _node_mlp1(xp, W1p, b1):
    return pl.pallas_call(
        _mlp1_body,
        grid=(_G,),
        in_specs=[
            pl.BlockSpec((_RB, 16), lambda i: (i, 0)),
            pl.BlockSpec((16, _HID), lambda i: (0, 0)),
            pl.BlockSpec((1, _HID), lambda i: (0, 0)),
        ],
        out_specs=pl.BlockSpec((_RB, _HID), lambda i: (i, 0)),
        out_shape=jax.ShapeDtypeStruct((_NP, _HID), _F32),
    )(xp, W1p, b1)


def _node_mlp2_pool(hp, mdp, bt3, W2, b2):
    return pl.pallas_call(
        _mlp2_pool_body,
        grid=(_G,),
        in_specs=[
            pl.BlockSpec((_RB, _HID), lambda i: (i, 0)),
            pl.BlockSpec((_RB, _MD), lambda i: (i, 0)),
            pl.BlockSpec((1, 1, _RB), lambda i: (i, 0, 0)),
            pl.BlockSpec((_HID, _HID), lambda i: (0, 0)),
            pl.BlockSpec((1, _HID), lambda i: (0, 0)),
        ],
        out_specs=[
            pl.BlockSpec((_B, _HID), lambda i: (0, 0)),
            pl.BlockSpec((_B, 128), lambda i: (0, 0)),
        ],
        out_shape=[
            jax.ShapeDtypeStruct((_B, _HID), _F32),
            jax.ShapeDtypeStruct((_B, 128), _F32),
        ],
    )(hp, mdp, bt3, W2, b2)


def _head(sums, cnt, Wp, bp, Wc, bc):
    return pl.pallas_call(
        _head_body,
        in_specs=[
            pl.BlockSpec((_B, _HID), lambda: (0, 0)),
            pl.BlockSpec((_B, 128), lambda: (0, 0)),
            pl.BlockSpec((_HID, _PROJ), lambda: (0, 0)),
            pl.BlockSpec((1, _PROJ), lambda: (0, 0)),
            pl.BlockSpec((_PROJ, _NCLS), lambda: (0, 0)),
            pl.BlockSpec((1, _NCLS), lambda: (0, 0)),
        ],
        out_specs=pl.BlockSpec((_B, _NCLS), lambda: (0, 0)),
        out_shape=jax.ShapeDtypeStruct((_B, _NCLS), _F32),
    )(sums, cnt, Wp, bp, Wc, bc)


def kernel(x, edge_index, batch, W1, b1, W2, b2, Wp, bp, Wc, bc):
    xp = jnp.zeros((_NP, 16), _F32).at[:_N, :11].set(x)
    W1p = jnp.zeros((16, _HID), _F32).at[:11, :].set(W1)
    hp = _node_mlp1(xp, W1p, b1.reshape(1, _HID))

    src = edge_index[0]
    dst = edge_index[1]
    h = hp[:_N]
    msg = jax.ops.segment_sum(h[src], dst, num_segments=_N)
    deg = jax.ops.segment_sum(jnp.ones((_E,), _F32), dst, num_segments=_N)
    md = jnp.concatenate([msg, deg[:, None], jnp.zeros((_N, _MD - _HID - 1), _F32)], axis=1)
    mdp = jnp.zeros((_NP, _MD), _F32).at[:_N].set(md)

    btp = jnp.full((_NP,), _B, jnp.int32).at[:_N].set(batch).reshape(_G, 1, _RB)
    sums, cnt = _node_mlp2_pool(hp, mdp, btp, W2, b2.reshape(1, _HID))
    logits = _head(sums, cnt, Wp, bp.reshape(1, _PROJ), Wc, bc.reshape(1, _NCLS))
    return jnp.broadcast_to(logits[None], (_SEQ, _B, _NCLS))


# speedup vs baseline: 4.2657x; 4.1428x over previous
import functools

import jax
import jax.numpy as jnp
from jax import lax
from jax.experimental import pallas as pl
from jax.experimental.pallas import tpu as pltpu
from jax.experimental.pallas import tpu_sc as plsc

_N = 50000
_E = 800000
_B = 64
_SEQ = 128
_HID = 256
_PROJ = 512
_NCLS = 1000

_RB = 1024
_NP = 51200
_G = _NP // _RB
_MD = 272

_NCORE = 2
_NSUB = 16
_RPP = _NP // 16
_NPASS = 8
_TRASH = 128
_SROWS = _RPP + _TRASH
_EW = _E // _NSUB
_WIN = 2000
_NWIN = _EW // _WIN
_KB = 128
_SEL = 2176
_ZR = 104

_F32 = jnp.float32
_I32 = jnp.int32


def _mlp1_body(x_ref, w_ref, b_ref, o_ref):
    h = jnp.dot(x_ref[...], w_ref[...], preferred_element_type=_F32)
    o_ref[:, :_HID] = jnp.maximum(h + b_ref[...], 0.0)
    lane = lax.broadcasted_iota(_I32, (_RB, _MD - _HID), 1)
    o_ref[:, _HID:] = jnp.where(lane == 0, 1.0, 0.0).astype(_F32)


def _mlp2_pool_body(hp_ref, md_ref, bt_ref, w_ref, b_ref, sums_ref, cnt_ref):
    i = pl.program_id(0)
    msg = md_ref[:, :_HID]
    deg = md_ref[:, _HID:_HID + 1]
    m = msg / jnp.maximum(deg, 1.0)
    Hb = jnp.dot(hp_ref[:, :_HID] + m, w_ref[...], preferred_element_type=_F32)
    Hb = jnp.maximum(Hb + b_ref[...], 0.0)
    bt = bt_ref[0]
    seg = lax.broadcasted_iota(_I32, (_B, _RB), 0)
    mask = (seg == bt).astype(_F32)
    psum = jnp.dot(mask, Hb, preferred_element_type=_F32)
    pcnt = jnp.sum(mask, axis=1, keepdims=True)

    @pl.when(i == 0)
    def _init():
        sums_ref[...] = psum
        cnt_ref[...] = jnp.broadcast_to(pcnt, (_B, 128))

    @pl.when(i > 0)
    def _acc():
        sums_ref[...] += psum
        cnt_ref[...] += jnp.broadcast_to(pcnt, (_B, 128))


def _head_body(sums_ref, cnt_ref, wp_ref, bp_ref, wc_ref, bc_ref, o_ref):
    cnt = cnt_ref[:, 0:1]
    hag = sums_ref[...] / jnp.maximum(cnt, 1.0)
    t = jnp.dot(hag, wp_ref[...], preferred_element_type=_F32) + bp_ref[...]
    o_ref[...] = jnp.dot(t, wc_ref[...], preferred_element_type=_F32) + bc_ref[...]


def _node_mlp1(xp, W1p, b1):
    return pl.pallas_call(
        _mlp1_body,
        grid=(_G,),
        in_specs=[
            pl.BlockSpec((_RB, 16), lambda i: (i, 0)),
            pl.BlockSpec((16, _HID), lambda i: (0, 0)),
            pl.BlockSpec((1, _HID), lambda i: (0, 0)),
        ],
        out_specs=pl.BlockSpec((_RB, _MD), lambda i: (i, 0)),
        out_shape=jax.ShapeDtypeStruct((_NP, _MD), _F32),
    )(xp, W1p, b1)


def _node_mlp2_pool(hp, mdp, bt3, W2, b2):
    return pl.pallas_call(
        _mlp2_pool_body,
        grid=(_G,),
        in_specs=[
            pl.BlockSpec((_RB, _MD), lambda i: (i, 0)),
            pl.BlockSpec((_RB, _MD), lambda i: (i, 0)),
            pl.BlockSpec((1, 1, _RB), lambda i: (i, 0, 0)),
            pl.BlockSpec((_HID, _HID), lambda i: (0, 0)),
            pl.BlockSpec((1, _HID), lambda i: (0, 0)),
        ],
        out_specs=[
            pl.BlockSpec((_B, _HID), lambda i: (0, 0)),
            pl.BlockSpec((_B, 128), lambda i: (0, 0)),
        ],
        out_shape=[
            jax.ShapeDtypeStruct((_B, _HID), _F32),
            jax.ShapeDtypeStruct((_B, 128), _F32),
        ],
    )(hp, mdp, bt3, W2, b2)


def _head(sums, cnt, Wp, bp, Wc, bc):
    return pl.pallas_call(
        _head_body,
        in_specs=[
            pl.BlockSpec((_B, _HID), lambda: (0, 0)),
            pl.BlockSpec((_B, 128), lambda: (0, 0)),
            pl.BlockSpec((_HID, _PROJ), lambda: (0, 0)),
            pl.BlockSpec((1, _PROJ), lambda: (0, 0)),
            pl.BlockSpec((_PROJ, _NCLS), lambda: (0, 0)),
            pl.BlockSpec((1, _NCLS), lambda: (0, 0)),
        ],
        out_specs=pl.BlockSpec((_B, _NCLS), lambda: (0, 0)),
        out_shape=jax.ShapeDtypeStruct((_B, _NCLS), _F32),
    )(sums, cnt, Wp, bp, Wc, bc)


def _sc_body(src_hbm, dst_hbm, hp_hbm, out_hbm,
             srcw, dstw, sel_src, sel_loc, srcb, locb, rows, zbuf,
             acc, esem1, esem2, gsem, zsem):
    c = lax.axis_index("c")
    s = lax.axis_index("s")
    lanes = lax.broadcasted_iota(_I32, (16,), 0)
    pad_src = lanes * 8
    pad_loc = _RPP + lanes

    def _zb(i, _):
        r = i // 17
        k = i - r * 17
        zbuf[r, pl.ds(k * 16, 16)] = jnp.zeros((16,), _F32)
        return 0
    lax.fori_loop(0, _ZR * 17, _zb, 0)

    def _fire(j, cnt):
        for k in range(_KB // 16):
            srcb[pl.ds(k * 16, 16)] = sel_src[pl.ds(j * _KB + k * 16, 16)]
            locb[pl.ds(k * 16, 16)] = sel_loc[pl.ds(j * _KB + k * 16, 16)]
        pltpu.async_copy(hp_hbm.at[srcb], rows, gsem).wait()
        pltpu.sync_copy(rows, acc.at[locb], add=True)

    def _pass(p, _):
        base = (2 * p + c) * _RPP

        plsc.subcore_barrier()
        z0 = s * (2 * _ZR)
        cz0 = pltpu.async_copy(zbuf, acc.at[pl.ds(z0, _ZR)], zsem)
        cz1 = pltpu.async_copy(zbuf, acc.at[pl.ds(z0 + _ZR, _ZR)], zsem)
        cz0.wait(); cz1.wait()
        plsc.subcore_barrier()

        def _window(w, cnt):
            e0 = s * _EW + w * _WIN
            c1 = pltpu.async_copy(src_hbm.at[pl.ds(e0, _WIN)], srcw, esem1)
            c2 = pltpu.async_copy(dst_hbm.at[pl.ds(e0, _WIN)], dstw, esem2)
            c1.wait(); c2.wait()

            def _compact(i, cnt):
                d = dstw[pl.ds(i * 16, 16)]
                sv = srcw[pl.ds(i * 16, 16)]
                loc = d - base
                m = (loc >= 0) & (loc < _RPP)
                mi = jnp.where(m, jnp.int32(1), jnp.int32(0))
                pos = plsc.cumsum(mi) - mi + cnt
                plsc.store_scatter(sel_loc, [pos], loc, mask=m)
                plsc.store_scatter(sel_src, [pos], sv, mask=m)
                return cnt + jnp.sum(mi)
            cnt = lax.fori_loop(0, _WIN // 16, _compact, cnt)

            for j in range(_SEL // _KB):
                @pl.when((j + 1) * _KB <= cnt)
                def _():
                    _fire(j, cnt)

            nfull = cnt // _KB
            roff = nfull * _KB
            rem = cnt - roff
            for k in range(_KB // 16):
                sv = sel_src[pl.ds(roff + k * 16, 16)]
                lv = sel_loc[pl.ds(roff + k * 16, 16)]
                sel_src[pl.ds(k * 16, 16)] = sv
                sel_loc[pl.ds(k * 16, 16)] = lv
            return rem

        cnt = lax.fori_loop(0, _NWIN, _window, jnp.int32(0))

        for k in range(_KB // 16):
            sel_src[pl.ds(cnt + k * 16, 16)] = pad_src
            sel_loc[pl.ds(cnt + k * 16, 16)] = pad_loc

        @pl.when(cnt > 0)
        def _():
            _fire(0, cnt)

        plsc.subcore_barrier()
        rb = _RPP // _NSUB
        pltpu.sync_copy(acc.at[pl.ds(s * rb, rb)],
                        out_hbm.at[pl.ds(base + s * rb, rb)])
        return 0

    lax.fori_loop(0, _NPASS, _pass, 0)


@functools.partial(jax.jit, static_argnums=())
def _sc_msgdeg(src, dst, hp):
    mesh = plsc.VectorSubcoreMesh(core_axis_name="c", subcore_axis_name="s")
    f = pl.kernel(
        _sc_body,
        out_type=jax.ShapeDtypeStruct((_NP, _MD), _F32),
        mesh=mesh,
        compiler_params=pltpu.CompilerParams(needs_layout_passes=False,
                                             use_tc_tiling_on_sc=False),
        scratch_types=[
            pltpu.VMEM((_WIN,), _I32),
            pltpu.VMEM((_WIN,), _I32),
            pltpu.VMEM((_SEL,), _I32),
            pltpu.VMEM((_SEL,), _I32),
            pltpu.VMEM((_KB,), _I32),
            pltpu.VMEM((_KB,), _I32),
            pltpu.VMEM((_KB, _MD), _F32),
            pltpu.VMEM((_ZR, _MD), _F32),
            pltpu.VMEM_SHARED((_SROWS, _MD), _F32),
            pltpu.SemaphoreType.DMA,
            pltpu.SemaphoreType.DMA,
            pltpu.SemaphoreType.DMA,
            pltpu.SemaphoreType.DMA,
        ],
    )
    return f(src, dst, hp)


def kernel(x, edge_index, batch, W1, b1, W2, b2, Wp, bp, Wc, bc):
    xp = jnp.zeros((_NP, 16), _F32).at[:_N, :11].set(x)
    W1p = jnp.zeros((16, _HID), _F32).at[:11, :].set(W1)
    hp = _node_mlp1(xp, W1p, b1.reshape(1, _HID))

    mdp = _sc_msgdeg(edge_index[0], edge_index[1], hp)

    btp = jnp.full((_NP,), _B, _I32).at[:_N].set(batch).reshape(_G, 1, _RB)
    sums, cnt = _node_mlp2_pool(hp, mdp, btp, W2, b2.reshape(1, _HID))
    logits = _head(sums, cnt, Wp, bp.reshape(1, _PROJ), Wc, bc.reshape(1, _NCLS))
    return jnp.broadcast_to(logits[None], (_SEQ, _B, _NCLS))


# speedup vs baseline: 4.4313x; 1.0388x over previous
import functools

import jax
import jax.numpy as jnp
from jax import lax
from jax.experimental import pallas as pl
from jax.experimental.pallas import tpu as pltpu
from jax.experimental.pallas import tpu_sc as plsc

_N = 50000
_E = 800000
_B = 64
_SEQ = 128
_HID = 256
_PROJ = 512
_NCLS = 1000

_RB = 1024
_NP = 51200
_G = _NP // _RB
_MD = 272

_NCORE = 2
_NSUB = 16
_RPP = _NP // 20
_NPASS = 10
_SROWS = _RPP
_EW = _E // _NSUB
_WIN = 2000
_NWIN = _EW // _WIN
_KB = 128
_SEL = 2176
_ZSH = _SROWS // _NSUB
_ZR = 16

_F32 = jnp.float32
_I32 = jnp.int32


def _mlp1_body(x_ref, w_ref, b_ref, o_ref):
    i = pl.program_id(0)
    h = jnp.dot(x_ref[...], w_ref[...], preferred_element_type=_F32)
    live = (i * _RB + lax.broadcasted_iota(_I32, (_RB, 1), 0)) < _N
    o_ref[:, :_HID] = jnp.where(live, jnp.maximum(h + b_ref[...], 0.0), 0.0)
    lane = lax.broadcasted_iota(_I32, (_RB, _MD - _HID), 1)
    o_ref[:, _HID:] = jnp.where(live & (lane == 0), 1.0, 0.0).astype(_F32)


def _mlp2_pool_body(hp_ref, md_ref, bt_ref, w_ref, b_ref, sums_ref, cnt_ref):
    i = pl.program_id(0)
    msg = md_ref[:, :_HID]
    deg = md_ref[:, _HID:_HID + 1]
    m = msg / jnp.maximum(deg, 1.0)
    Hb = jnp.dot(hp_ref[:, :_HID] + m, w_ref[...], preferred_element_type=_F32)
    Hb = jnp.maximum(Hb + b_ref[...], 0.0)
    bt = bt_ref[0]
    seg = lax.broadcasted_iota(_I32, (_B, _RB), 0)
    mask = (seg == bt).astype(_F32)
    psum = jnp.dot(mask, Hb, preferred_element_type=_F32)
    pcnt = jnp.sum(mask, axis=1, keepdims=True)

    @pl.when(i == 0)
    def _init():
        sums_ref[...] = psum
        cnt_ref[...] = jnp.broadcast_to(pcnt, (_B, 128))

    @pl.when(i > 0)
    def _acc():
        sums_ref[...] += psum
        cnt_ref[...] += jnp.broadcast_to(pcnt, (_B, 128))


def _head_body(sums_ref, cnt_ref, wp_ref, bp_ref, wc_ref, bc_ref, o_ref):
    cnt = cnt_ref[:, 0:1]
    hag = sums_ref[...] / jnp.maximum(cnt, 1.0)
    t = jnp.dot(hag, wp_ref[...], preferred_element_type=_F32) + bp_ref[...]
    o_ref[...] = jnp.dot(t, wc_ref[...], preferred_element_type=_F32) + bc_ref[...]


def _node_mlp1(xp, W1p, b1):
    return pl.pallas_call(
        _mlp1_body,
        grid=(_G,),
        in_specs=[
            pl.BlockSpec((_RB, 16), lambda i: (i, 0)),
            pl.BlockSpec((16, _HID), lambda i: (0, 0)),
            pl.BlockSpec((1, _HID), lambda i: (0, 0)),
        ],
        out_specs=pl.BlockSpec((_RB, _MD), lambda i: (i, 0)),
        out_shape=jax.ShapeDtypeStruct((_NP, _MD), _F32),
    )(xp, W1p, b1)


def _node_mlp2_pool(hp, mdp, bt3, W2, b2):
    return pl.pallas_call(
        _mlp2_pool_body,
        grid=(_G,),
        in_specs=[
            pl.BlockSpec((_RB, _MD), lambda i: (i, 0)),
            pl.BlockSpec((_RB, _MD), lambda i: (i, 0)),
            pl.BlockSpec((1, 1, _RB), lambda i: (i, 0, 0)),
            pl.BlockSpec((_HID, _HID), lambda i: (0, 0)),
            pl.BlockSpec((1, _HID), lambda i: (0, 0)),
        ],
        out_specs=[
            pl.BlockSpec((_B, _HID), lambda i: (0, 0)),
            pl.BlockSpec((_B, 128), lambda i: (0, 0)),
        ],
        out_shape=[
            jax.ShapeDtypeStruct((_B, _HID), _F32),
            jax.ShapeDtypeStruct((_B, 128), _F32),
        ],
    )(hp, mdp, bt3, W2, b2)


def _head(sums, cnt, Wp, bp, Wc, bc):
    return pl.pallas_call(
        _head_body,
        in_specs=[
            pl.BlockSpec((_B, _HID), lambda: (0, 0)),
            pl.BlockSpec((_B, 128), lambda: (0, 0)),
            pl.BlockSpec((_HID, _PROJ), lambda: (0, 0)),
            pl.BlockSpec((1, _PROJ), lambda: (0, 0)),
            pl.BlockSpec((_PROJ, _NCLS), lambda: (0, 0)),
            pl.BlockSpec((1, _NCLS), lambda: (0, 0)),
        ],
        out_specs=pl.BlockSpec((_B, _NCLS), lambda: (0, 0)),
        out_shape=jax.ShapeDtypeStruct((_B, _NCLS), _F32),
    )(sums, cnt, Wp, bp, Wc, bc)


def _sc_body(src_hbm, dst_hbm, hp_hbm, out_hbm,
             srcw0, dstw0, srcw1, dstw1, sel_src, sel_loc,
             srcb0, locb0, srcb1, locb1, rows0, rows1, zbuf,
             acc, esemA, esemB, gsem0, gsem1, zsem):
    c = lax.axis_index("c")
    s = lax.axis_index("s")
    lanes = lax.broadcasted_iota(_I32, (16,), 0)
    pad_src = _N + lanes * 8
    pad_loc = lanes
    srcbs, locbs, rowss, gsems = (srcb0, srcb1), (locb0, locb1), (rows0, rows1), (gsem0, gsem1)

    def _zb(i, _):
        r = i // 17
        k = i - r * 17
        zbuf[r, pl.ds(k * 16, 16)] = jnp.zeros((16,), _F32)
        return 0
    lax.fori_loop(0, _ZR * 17, _zb, 0)

    def _stage_gather(t, j):
        for k in range(_KB // 16):
            srcbs[t][pl.ds(k * 16, 16)] = sel_src[pl.ds(j * _KB + k * 16, 16)]
            locbs[t][pl.ds(k * 16, 16)] = sel_loc[pl.ds(j * _KB + k * 16, 16)]
        pltpu.async_copy(hp_hbm.at[srcbs[t]], rowss[t], gsems[t])

    def _finish_scatter(t):
        pltpu.make_async_copy(hp_hbm.at[srcbs[t]], rowss[t], gsems[t]).wait()
        pltpu.sync_copy(rowss[t], acc.at[locbs[t]], add=True)

    def _pass(p, _):
        base = (2 * p + c) * _RPP

        plsc.subcore_barrier()
        z0 = s * _ZSH
        zds = [pltpu.async_copy(zbuf, acc.at[pl.ds(z0 + _ZR * k, _ZR)], zsem)
               for k in range(_ZSH // _ZR)]
        for d in zds:
            d.wait()
        plsc.subcore_barrier()

        def _process(sw, dw, cnt):
            def _compact(i, cnt):
                d = dw[pl.ds(i * 16, 16)]
                sv = sw[pl.ds(i * 16, 16)]
                loc = d - base
                m = (loc >= 0) & (loc < _RPP)
                mi = jnp.where(m, jnp.int32(1), jnp.int32(0))
                pos = plsc.cumsum(mi) - mi + cnt
                plsc.store_scatter(sel_loc, [pos], loc, mask=m)
                plsc.store_scatter(sel_src, [pos], sv, mask=m)
                return cnt + jnp.sum(mi)
            cnt = lax.fori_loop(0, _WIN // 16, _compact, cnt)

            for g in range(_SEL // _KB // 2):
                for t in range(2):
                    j = 2 * g + t
                    @pl.when((j + 1) * _KB <= cnt)
                    def _():
                        _stage_gather(t, j)
                for t in range(2):
                    j = 2 * g + t
                    @pl.when((j + 1) * _KB <= cnt)
                    def _():
                        _finish_scatter(t)

            roff = (cnt // _KB) * _KB
            for k in range(_KB // 16):
                sv = sel_src[pl.ds(roff + k * 16, 16)]
                lv = sel_loc[pl.ds(roff + k * 16, 16)]
                sel_src[pl.ds(k * 16, 16)] = sv
                sel_loc[pl.ds(k * 16, 16)] = lv
            return cnt - roff

        def _issue(w, sw, dw, sem):
            e0 = s * _EW + w * _WIN
            pltpu.async_copy(src_hbm.at[pl.ds(e0, _WIN)], sw, sem)
            pltpu.async_copy(dst_hbm.at[pl.ds(e0, _WIN)], dw, sem)

        def _drain(sw, dw, sem):
            pltpu.make_async_copy(src_hbm.at[pl.ds(0, _WIN)], sw, sem).wait()
            pltpu.make_async_copy(src_hbm.at[pl.ds(0, _WIN)], dw, sem).wait()

        _issue(0, srcw0, dstw0, esemA)

        def _pair(w2, cnt):
            w = 2 * w2
            _issue(w + 1, srcw1, dstw1, esemB)
            _drain(srcw0, dstw0, esemA)
            cnt = _process(srcw0, dstw0, cnt)
            _issue(w + 2, srcw0, dstw0, esemA)
            _drain(srcw1, dstw1, esemB)
            cnt = _process(srcw1, dstw1, cnt)
            return cnt

        cnt = lax.fori_loop(0, (_NWIN - 1) // 2, _pair, jnp.int32(0))
        _drain(srcw0, dstw0, esemA)
        cnt = _process(srcw0, dstw0, cnt)

        for k in range(_KB // 16):
            sel_src[pl.ds(cnt + k * 16, 16)] = pad_src
            sel_loc[pl.ds(cnt + k * 16, 16)] = pad_loc

        @pl.when(cnt > 0)
        def _():
            _stage_gather(0, 0)
            _finish_scatter(0)

        plsc.subcore_barrier()
        rb = _RPP // _NSUB
        pltpu.sync_copy(acc.at[pl.ds(s * rb, rb)],
                        out_hbm.at[pl.ds(base + s * rb, rb)])
        return 0

    lax.fori_loop(0, _NPASS, _pass, 0)


@functools.partial(jax.jit, static_argnums=())
def _sc_msgdeg(src, dst, hp):
    mesh = plsc.VectorSubcoreMesh(core_axis_name="c", subcore_axis_name="s")
    f = pl.kernel(
        _sc_body,
        out_type=jax.ShapeDtypeStruct((_NP, _MD), _F32),
        mesh=mesh,
        compiler_params=pltpu.CompilerParams(needs_layout_passes=False,
                                             use_tc_tiling_on_sc=False),
        scratch_types=[
            pltpu.VMEM((_WIN,), _I32),
            pltpu.VMEM((_WIN,), _I32),
            pltpu.VMEM((_WIN,), _I32),
            pltpu.VMEM((_WIN,), _I32),
            pltpu.VMEM((_SEL,), _I32),
            pltpu.VMEM((_SEL,), _I32),
            pltpu.VMEM((_KB,), _I32),
            pltpu.VMEM((_KB,), _I32),
            pltpu.VMEM((_KB,), _I32),
            pltpu.VMEM((_KB,), _I32),
            pltpu.VMEM((_KB, _MD), _F32),
            pltpu.VMEM((_KB, _MD), _F32),
            pltpu.VMEM((_ZR, _MD), _F32),
            pltpu.VMEM_SHARED((_SROWS, _MD), _F32),
            pltpu.SemaphoreType.DMA,
            pltpu.SemaphoreType.DMA,
            pltpu.SemaphoreType.DMA,
            pltpu.SemaphoreType.DMA,
            pltpu.SemaphoreType.DMA,
        ],
    )
    return f(src, dst, hp)


def kernel(x, edge_index, batch, W1, b1, W2, b2, Wp, bp, Wc, bc):
    xp = jnp.zeros((_NP, 16), _F32).at[:_N, :11].set(x)
    W1p = jnp.zeros((16, _HID), _F32).at[:11, :].set(W1)
    hp = _node_mlp1(xp, W1p, b1.reshape(1, _HID))

    mdp = _sc_msgdeg(edge_index[0], edge_index[1], hp)

    btp = jnp.full((_NP,), _B, _I32).at[:_N].set(batch).reshape(_G, 1, _RB)
    sums, cnt = _node_mlp2_pool(hp, mdp, btp, W2, b2.reshape(1, _HID))
    logits = _head(sums, cnt, Wp, bp.reshape(1, _PROJ), Wc, bc.reshape(1, _NCLS))
    return jnp.broadcast_to(logits[None], (_SEQ, _B, _NCLS))


# speedup vs baseline: 5.2410x; 1.1827x over previous
import functools

import jax
import jax.numpy as jnp
from jax import lax
from jax.experimental import pallas as pl
from jax.experimental.pallas import tpu as pltpu
from jax.experimental.pallas import tpu_sc as plsc

_N = 50000
_E = 800000
_B = 64
_SEQ = 128
_HID = 256
_PROJ = 512
_NCLS = 1000

_RB = 1024
_NP = 51200
_G = _NP // _RB
_MD = 272

_NCORE = 2
_NSUB = 16
_RPP = _NP // 20
_NPASS = 10
_SROWS = _RPP
_EW = _E // _NSUB
_WIN = 2000
_NWIN = _EW // _WIN
_KB = 128
_SEL = 2176
_ZSH = _SROWS // _NSUB
_ZR = 16

_F32 = jnp.float32
_I32 = jnp.int32


def _mlp1_body(x_ref, w_ref, b_ref, o_ref):
    i = pl.program_id(0)
    h = jnp.dot(x_ref[...], w_ref[...], preferred_element_type=_F32)
    live = (i * _RB + lax.broadcasted_iota(_I32, (_RB, 1), 0)) < _N
    o_ref[:, :_HID] = jnp.where(live, jnp.maximum(h + b_ref[...], 0.0), 0.0)
    lane = lax.broadcasted_iota(_I32, (_RB, _MD - _HID), 1)
    o_ref[:, _HID:] = jnp.where(live & (lane == 0), 1.0, 0.0).astype(_F32)


def _mlp2_pool_body(hp_ref, md_ref, bt_ref, w_ref, b_ref, sums_ref, cnt_ref):
    i = pl.program_id(0)
    msg = md_ref[:, :_HID]
    deg = md_ref[:, _HID:_HID + 1]
    m = msg / jnp.maximum(deg, 1.0)
    Hb = jnp.dot(hp_ref[:, :_HID] + m, w_ref[...], preferred_element_type=_F32)
    Hb = jnp.maximum(Hb + b_ref[...], 0.0)
    bt = bt_ref[0]
    seg = lax.broadcasted_iota(_I32, (_B, _RB), 0)
    mask = (seg == bt).astype(_F32)
    psum = jnp.dot(mask, Hb, preferred_element_type=_F32)
    pcnt = jnp.sum(mask, axis=1, keepdims=True)

    @pl.when(i == 0)
    def _init():
        sums_ref[...] = psum
        cnt_ref[...] = jnp.broadcast_to(pcnt, (_B, 128))

    @pl.when(i > 0)
    def _acc():
        sums_ref[...] += psum
        cnt_ref[...] += jnp.broadcast_to(pcnt, (_B, 128))


def _head_body(sums_ref, cnt_ref, wp_ref, bp_ref, wc_ref, bc_ref, o_ref):
    cnt = cnt_ref[:, 0:1]
    hag = sums_ref[...] / jnp.maximum(cnt, 1.0)
    t = jnp.dot(hag, wp_ref[...], preferred_element_type=_F32) + bp_ref[...]
    o_ref[...] = jnp.dot(t, wc_ref[...], preferred_element_type=_F32) + bc_ref[...]


def _node_mlp1(xp, W1p, b1):
    return pl.pallas_call(
        _mlp1_body,
        grid=(_G,),
        in_specs=[
            pl.BlockSpec((_RB, 16), lambda i: (i, 0)),
            pl.BlockSpec((16, _HID), lambda i: (0, 0)),
            pl.BlockSpec((1, _HID), lambda i: (0, 0)),
        ],
        out_specs=pl.BlockSpec((_RB, _MD), lambda i: (i, 0)),
        out_shape=jax.ShapeDtypeStruct((_NP, _MD), _F32),
    )(xp, W1p, b1)


def _node_mlp2_pool(hp, mdp, bt3, W2, b2):
    return pl.pallas_call(
        _mlp2_pool_body,
        grid=(_G,),
        in_specs=[
            pl.BlockSpec((_RB, _MD), lambda i: (i, 0)),
            pl.BlockSpec((_RB, _MD), lambda i: (i, 0)),
            pl.BlockSpec((1, 1, _RB), lambda i: (i, 0, 0)),
            pl.BlockSpec((_HID, _HID), lambda i: (0, 0)),
            pl.BlockSpec((1, _HID), lambda i: (0, 0)),
        ],
        out_specs=[
            pl.BlockSpec((_B, _HID), lambda i: (0, 0)),
            pl.BlockSpec((_B, 128), lambda i: (0, 0)),
        ],
        out_shape=[
            jax.ShapeDtypeStruct((_B, _HID), _F32),
            jax.ShapeDtypeStruct((_B, 128), _F32),
        ],
    )(hp, mdp, bt3, W2, b2)


def _head(sums, cnt, Wp, bp, Wc, bc):
    return pl.pallas_call(
        _head_body,
        in_specs=[
            pl.BlockSpec((_B, _HID), lambda: (0, 0)),
            pl.BlockSpec((_B, 128), lambda: (0, 0)),
            pl.BlockSpec((_HID, _PROJ), lambda: (0, 0)),
            pl.BlockSpec((1, _PROJ), lambda: (0, 0)),
            pl.BlockSpec((_PROJ, _NCLS), lambda: (0, 0)),
            pl.BlockSpec((1, _NCLS), lambda: (0, 0)),
        ],
        out_specs=pl.BlockSpec((_B, _NCLS), lambda: (0, 0)),
        out_shape=jax.ShapeDtypeStruct((_B, _NCLS), _F32),
    )(sums, cnt, Wp, bp, Wc, bc)


def _sc_body(src_hbm, dst_hbm, hp_hbm, out_hbm,
             srcw0, dstw0, srcw1, dstw1, sel_src, sel_loc,
             srcb0, locb0, srcb1, locb1, rows0, rows1, zbuf,
             acc, esemA, esemB, gsem0, gsem1, ssem0, ssem1, zsem):
    c = lax.axis_index("c")
    s = lax.axis_index("s")
    lanes = lax.broadcasted_iota(_I32, (16,), 0)
    pad_src = _N + lanes * 8
    pad_loc = lanes
    srcbs, locbs, rowss = (srcb0, srcb1), (locb0, locb1), (rows0, rows1)
    gsems, ssems = (gsem0, gsem1), (ssem0, ssem1)

    def _zb(i, _):
        r = i // 17
        k = i - r * 17
        zbuf[r, pl.ds(k * 16, 16)] = jnp.zeros((16,), _F32)
        return 0
    lax.fori_loop(0, _ZR * 17, _zb, 0)

    def _stage_gather(t, j):
        for k in range(_KB // 16):
            srcbs[t][pl.ds(k * 16, 16)] = sel_src[pl.ds(j * _KB + k * 16, 16)]
            locbs[t][pl.ds(k * 16, 16)] = sel_loc[pl.ds(j * _KB + k * 16, 16)]
        pltpu.async_copy(hp_hbm.at[srcbs[t]], rowss[t], gsems[t])

    def _wait_gather(t):
        pltpu.make_async_copy(hp_hbm.at[srcbs[t]], rowss[t], gsems[t]).wait()

    def _issue_scatter(t):
        pltpu.async_copy(rowss[t], acc.at[locbs[t]], ssems[t], add=True)

    def _drain_scatter(t):
        pltpu.make_async_copy(rowss[t], acc.at[locbs[t]], ssems[t]).wait()

    def _pass(p, _):
        base = (2 * p + c) * _RPP

        plsc.subcore_barrier()
        z0 = s * _ZSH
        zds = [pltpu.async_copy(zbuf, acc.at[pl.ds(z0 + _ZR * k, _ZR)], zsem)
               for k in range(_ZSH // _ZR)]
        for d in zds:
            d.wait()
        plsc.subcore_barrier()

        def _process(sw, dw, carry):
            cnt, p0, p1 = carry
            pends = (p0, p1)

            def _compact(i, cv):
                d = dw[pl.ds(i * 16, 16)]
                sv = sw[pl.ds(i * 16, 16)]
                loc = d - base
                m = (loc >= 0) & (loc < _RPP)
                mi = jnp.where(m, jnp.int32(1), jnp.int32(0))
                pos = plsc.cumsum(mi) - mi + cv
                plsc.store_scatter(sel_loc, [pos], loc, mask=m)
                plsc.store_scatter(sel_src, [pos], sv, mask=m)
                return cv + plsc.all_reduce_population_count(m)
            cnt_v = lax.fori_loop(0, _WIN // 16, _compact,
                                  jnp.broadcast_to(cnt, (16,)))
            cnt = jnp.max(cnt_v)

            for g in range(_SEL // _KB // 2):
                for t in range(2):
                    j = 2 * g + t
                    @pl.when((j + 1) * _KB <= cnt)
                    def _():
                        if j >= 2:
                            _drain_scatter(t)
                        else:
                            @pl.when(pends[t] > 0)
                            def _():
                                _drain_scatter(t)
                        _stage_gather(t, j)
                for t in range(2):
                    j = 2 * g + t
                    @pl.when((j + 1) * _KB <= cnt)
                    def _():
                        _wait_gather(t)
                        _issue_scatter(t)

            nf = cnt // _KB
            roff = nf * _KB
            for k in range(_KB // 16):
                sv = sel_src[pl.ds(roff + k * 16, 16)]
                lv = sel_loc[pl.ds(roff + k * 16, 16)]
                sel_src[pl.ds(k * 16, 16)] = sv
                sel_loc[pl.ds(k * 16, 16)] = lv
            p0 = jnp.where(nf >= 1, jnp.int32(1), p0)
            p1 = jnp.where(nf >= 2, jnp.int32(1), p1)
            return cnt - roff, p0, p1

        def _issue(w, sw, dw, sem):
            e0 = s * _EW + w * _WIN
            pltpu.async_copy(src_hbm.at[pl.ds(e0, _WIN)], sw, sem)
            pltpu.async_copy(dst_hbm.at[pl.ds(e0, _WIN)], dw, sem)

        def _drain(sw, dw, sem):
            pltpu.make_async_copy(src_hbm.at[pl.ds(0, _WIN)], sw, sem).wait()
            pltpu.make_async_copy(src_hbm.at[pl.ds(0, _WIN)], dw, sem).wait()

        _issue(0, srcw0, dstw0, esemA)

        def _pair(w2, carry):
            w = 2 * w2
            _issue(w + 1, srcw1, dstw1, esemB)
            _drain(srcw0, dstw0, esemA)
            carry = _process(srcw0, dstw0, carry)
            _issue(w + 2, srcw0, dstw0, esemA)
            _drain(srcw1, dstw1, esemB)
            carry = _process(srcw1, dstw1, carry)
            return carry

        carry = lax.fori_loop(0, (_NWIN - 1) // 2, _pair,
                              (jnp.int32(0), jnp.int32(0), jnp.int32(0)))
        _drain(srcw0, dstw0, esemA)
        cnt, p0, p1 = _process(srcw0, dstw0, carry)

        @pl.when(p0 > 0)
        def _():
            _drain_scatter(0)

        @pl.when(p1 > 0)
        def _():
            _drain_scatter(1)

        for k in range(_KB // 16):
            sel_src[pl.ds(cnt + k * 16, 16)] = pad_src
            sel_loc[pl.ds(cnt + k * 16, 16)] = pad_loc

        @pl.when(cnt > 0)
        def _():
            _stage_gather(0, 0)
            _wait_gather(0)
            _issue_scatter(0)
            _drain_scatter(0)

        plsc.subcore_barrier()
        rb = _RPP // _NSUB
        pltpu.sync_copy(acc.at[pl.ds(s * rb, rb)],
                        out_hbm.at[pl.ds(base + s * rb, rb)])
        return 0

    lax.fori_loop(0, _NPASS, _pass, 0)


@functools.partial(jax.jit, static_argnums=())
def _sc_msgdeg(src, dst, hp):
    mesh = plsc.VectorSubcoreMesh(core_axis_name="c", subcore_axis_name="s")
    f = pl.kernel(
        _sc_body,
        out_type=jax.ShapeDtypeStruct((_NP, _MD), _F32),
        mesh=mesh,
        compiler_params=pltpu.CompilerParams(needs_layout_passes=False,
                                             use_tc_tiling_on_sc=False),
        scratch_types=[
            pltpu.VMEM((_WIN,), _I32),
            pltpu.VMEM((_WIN,), _I32),
            pltpu.VMEM((_WIN,), _I32),
            pltpu.VMEM((_WIN,), _I32),
            pltpu.VMEM((_SEL,), _I32),
            pltpu.VMEM((_SEL,), _I32),
            pltpu.VMEM((_KB,), _I32),
            pltpu.VMEM((_KB,), _I32),
            pltpu.VMEM((_KB,), _I32),
            pltpu.VMEM((_KB,), _I32),
            pltpu.VMEM((_KB, _MD), _F32),
            pltpu.VMEM((_KB, _MD), _F32),
            pltpu.VMEM((_ZR, _MD), _F32),
            pltpu.VMEM_SHARED((_SROWS, _MD), _F32),
            pltpu.SemaphoreType.DMA,
            pltpu.SemaphoreType.DMA,
            pltpu.SemaphoreType.DMA,
            pltpu.SemaphoreType.DMA,
            pltpu.SemaphoreType.DMA,
            pltpu.SemaphoreType.DMA,
            pltpu.SemaphoreType.DMA,
        ],
    )
    return f(src, dst, hp)


def kernel(x, edge_index, batch, W1, b1, W2, b2, Wp, bp, Wc, bc):
    xp = jnp.zeros((_NP, 16), _F32).at[:_N, :11].set(x)
    W1p = jnp.zeros((16, _HID), _F32).at[:11, :].set(W1)
    hp = _node_mlp1(xp, W1p, b1.reshape(1, _HID))

    mdp = _sc_msgdeg(edge_index[0], edge_index[1], hp)

    btp = jnp.full((_NP,), _B, _I32).at[:_N].set(batch).reshape(_G, 1, _RB)
    sums, cnt = _node_mlp2_pool(hp, mdp, btp, W2, b2.reshape(1, _HID))
    logits = _head(sums, cnt, Wp, bp.reshape(1, _PROJ), Wc, bc.reshape(1, _NCLS))
    return jnp.broadcast_to(logits[None], (_SEQ, _B, _NCLS))


# speedup vs baseline: 6.1423x; 1.1720x over previous
import functools

import jax
import jax.numpy as jnp
from jax import lax
from jax.experimental import pallas as pl
from jax.experimental.pallas import tpu as pltpu
from jax.experimental.pallas import tpu_sc as plsc

_N = 50000
_E = 800000
_B = 64
_SEQ = 128
_HID = 256
_PROJ = 512
_NCLS = 1000

_RB = 1024
_NP = 51200
_G = _NP // _RB
_MD = 272
_MD16 = 288

_NCORE = 2
_NSUB = 16
_RPP = _NP // 16
_NPASS = 8
_SROWS = _RPP
_EW = _E // _NSUB
_WIN = 2000
_NWIN = _EW // _WIN
_KB = 128
_NBUF = 2
_SEL = 2176
_ZSH = _SROWS // _NSUB
_ZR = 8

_F32 = jnp.float32
_BF16 = jnp.bfloat16
_I32 = jnp.int32


def _mlp1_body(x_ref, w_ref, b_ref, o_ref, o16_ref):
    i = pl.program_id(0)
    h = jnp.dot(x_ref[...], w_ref[...], preferred_element_type=_F32)
    live = (i * _RB + lax.broadcasted_iota(_I32, (_RB, 1), 0)) < _N
    hr = jnp.where(live, jnp.maximum(h + b_ref[...], 0.0), 0.0)
    o_ref[:, :_HID] = hr
    lane = lax.broadcasted_iota(_I32, (_RB, _MD - _HID), 1)
    ones = jnp.where(live & (lane == 0), 1.0, 0.0).astype(_F32)
    o_ref[:, _HID:] = ones
    o16_ref[:, :_HID] = hr.astype(_BF16)
    lane16 = lax.broadcasted_iota(_I32, (_RB, _MD16 - _HID), 1)
    o16_ref[:, _HID:] = jnp.where(live & (lane16 == 0), 1.0, 0.0).astype(_BF16)


def _mlp2_pool_body(hp_ref, md_ref, bt_ref, w_ref, b_ref, sums_ref, cnt_ref):
    i = pl.program_id(0)
    msg = md_ref[:, :_HID].astype(_F32)
    deg = md_ref[:, _HID:_HID + 1].astype(_F32)
    m = msg / jnp.maximum(deg, 1.0)
    Hb = jnp.dot(hp_ref[:, :_HID] + m, w_ref[...], preferred_element_type=_F32)
    Hb = jnp.maximum(Hb + b_ref[...], 0.0)
    bt = bt_ref[0]
    seg = lax.broadcasted_iota(_I32, (_B, _RB), 0)
    mask = (seg == bt).astype(_F32)
    psum = jnp.dot(mask, Hb, preferred_element_type=_F32)
    pcnt = jnp.sum(mask, axis=1, keepdims=True)

    @pl.when(i == 0)
    def _init():
        sums_ref[...] = psum
        cnt_ref[...] = jnp.broadcast_to(pcnt, (_B, 128))

    @pl.when(i > 0)
    def _acc():
        sums_ref[...] += psum
        cnt_ref[...] += jnp.broadcast_to(pcnt, (_B, 128))


def _head_body(sums_ref, cnt_ref, wp_ref, bp_ref, wc_ref, bc_ref, o_ref):
    cnt = cnt_ref[:, 0:1]
    hag = sums_ref[...] / jnp.maximum(cnt, 1.0)
    t = jnp.dot(hag, wp_ref[...], preferred_element_type=_F32) + bp_ref[...]
    o_ref[...] = jnp.dot(t, wc_ref[...], preferred_element_type=_F32) + bc_ref[...]


def _node_mlp1(xp, W1p, b1):
    return pl.pallas_call(
        _mlp1_body,
        grid=(_G,),
        in_specs=[
            pl.BlockSpec((_RB, 16), lambda i: (i, 0)),
            pl.BlockSpec((16, _HID), lambda i: (0, 0)),
            pl.BlockSpec((1, _HID), lambda i: (0, 0)),
        ],
        out_specs=[
            pl.BlockSpec((_RB, _MD), lambda i: (i, 0)),
            pl.BlockSpec((_RB, _MD16), lambda i: (i, 0)),
        ],
        out_shape=[
            jax.ShapeDtypeStruct((_NP, _MD), _F32),
            jax.ShapeDtypeStruct((_NP, _MD16), _BF16),
        ],
    )(xp, W1p, b1)


def _node_mlp2_pool(hp, mdp, bt3, W2, b2):
    return pl.pallas_call(
        _mlp2_pool_body,
        grid=(_G,),
        in_specs=[
            pl.BlockSpec((_RB, _MD), lambda i: (i, 0)),
            pl.BlockSpec((_RB, _MD16), lambda i: (i, 0)),
            pl.BlockSpec((1, 1, _RB), lambda i: (i, 0, 0)),
            pl.BlockSpec((_HID, _HID), lambda i: (0, 0)),
            pl.BlockSpec((1, _HID), lambda i: (0, 0)),
        ],
        out_specs=[
            pl.BlockSpec((_B, _HID), lambda i: (0, 0)),
            pl.BlockSpec((_B, 128), lambda i: (0, 0)),
        ],
        out_shape=[
            jax.ShapeDtypeStruct((_B, _HID), _F32),
            jax.ShapeDtypeStruct((_B, 128), _F32),
        ],
    )(hp, mdp, bt3, W2, b2)


def _head(sums, cnt, Wp, bp, Wc, bc):
    return pl.pallas_call(
        _head_body,
        in_specs=[
            pl.BlockSpec((_B, _HID), lambda: (0, 0)),
            pl.BlockSpec((_B, 128), lambda: (0, 0)),
            pl.BlockSpec((_HID, _PROJ), lambda: (0, 0)),
            pl.BlockSpec((1, _PROJ), lambda: (0, 0)),
            pl.BlockSpec((_PROJ, _NCLS), lambda: (0, 0)),
            pl.BlockSpec((1, _NCLS), lambda: (0, 0)),
        ],
        out_specs=pl.BlockSpec((_B, _NCLS), lambda: (0, 0)),
        out_shape=jax.ShapeDtypeStruct((_B, _NCLS), _F32),
    )(sums, cnt, Wp, bp, Wc, bc)


def _sc_body(src_hbm, dst_hbm, hp_hbm, out_hbm,
             srcw0, dstw0, srcw1, dstw1, sel_src, sel_loc,
             srcb0, locb0, srcb1, locb1, rows0, rows1, zbuf,
             acc, esemA, esemB, gsem0, gsem1, ssem0, ssem1, zsem):
    c = lax.axis_index("c")
    s = lax.axis_index("s")
    lanes = lax.broadcasted_iota(_I32, (16,), 0)
    pad_src = _N + lanes * 8
    pad_loc = lanes
    srcbs, locbs, rowss = (srcb0, srcb1), (locb0, locb1), (rows0, rows1)
    gsems, ssems = (gsem0, gsem1), (ssem0, ssem1)

    def _zb(i, _):
        r = i // (_MD16 // 32)
        k = i - r * (_MD16 // 32)
        zbuf[r, pl.ds(k * 32, 32)] = jnp.zeros((32,), _BF16)
        return 0
    lax.fori_loop(0, _ZR * (_MD16 // 32), _zb, 0)

    def _stage_gather(t, j):
        for k in range(_KB // 16):
            srcbs[t][pl.ds(k * 16, 16)] = sel_src[pl.ds(j * _KB + k * 16, 16)]
            locbs[t][pl.ds(k * 16, 16)] = sel_loc[pl.ds(j * _KB + k * 16, 16)]
        pltpu.async_copy(hp_hbm.at[srcbs[t]], rowss[t], gsems[t])

    def _wait_gather(t):
        pltpu.make_async_copy(hp_hbm.at[srcbs[t]], rowss[t], gsems[t]).wait()

    def _issue_scatter(t):
        pltpu.async_copy(rowss[t], acc.at[locbs[t]], ssems[t], add=True)

    def _drain_scatter(t):
        pltpu.make_async_copy(rowss[t], acc.at[locbs[t]], ssems[t]).wait()

    def _pass(p, _):
        base = (2 * p + c) * _RPP

        plsc.subcore_barrier()
        z0 = s * _ZSH
        zds = [pltpu.async_copy(zbuf, acc.at[pl.ds(z0 + _ZR * k, _ZR)], zsem)
               for k in range(_ZSH // _ZR)]
        for d in zds:
            d.wait()
        plsc.subcore_barrier()

        def _process(sw, dw, carry):
            cnt, p0, p1 = carry
            pends = (p0, p1)

            def _compact(i, cv):
                d = dw[pl.ds(i * 16, 16)]
                sv = sw[pl.ds(i * 16, 16)]
                loc = d - base
                m = (loc >= 0) & (loc < _RPP)
                mi = jnp.where(m, jnp.int32(1), jnp.int32(0))
                pos = plsc.cumsum(mi) - mi + cv
                plsc.store_scatter(sel_loc, [pos], loc, mask=m)
                plsc.store_scatter(sel_src, [pos], sv, mask=m)
                return cv + plsc.all_reduce_population_count(m)
            cnt_v = lax.fori_loop(0, _WIN // 16, _compact,
                                  jnp.broadcast_to(cnt, (16,)))
            cnt = jnp.max(cnt_v)

            nf = cnt // _KB

            def _grp(g, _):
                for t in range(_NBUF):
                    j = _NBUF * g + t
                    @pl.when(j < nf)
                    def _():
                        @pl.when((g > 0) | (pends[t] > 0))
                        def _():
                            _drain_scatter(t)
                        _stage_gather(t, j)
                for t in range(_NBUF):
                    j = _NBUF * g + t
                    @pl.when(j < nf)
                    def _():
                        _wait_gather(t)
                        _issue_scatter(t)
                return 0
            lax.fori_loop(0, (nf + _NBUF - 1) // _NBUF, _grp, 0)

            roff = nf * _KB
            for k in range(_KB // 16):
                sv = sel_src[pl.ds(roff + k * 16, 16)]
                lv = sel_loc[pl.ds(roff + k * 16, 16)]
                sel_src[pl.ds(k * 16, 16)] = sv
                sel_loc[pl.ds(k * 16, 16)] = lv
            p0 = jnp.where(nf >= 1, jnp.int32(1), p0)
            p1 = jnp.where(nf >= 2, jnp.int32(1), p1)
            return cnt - roff, p0, p1

        def _issue(w, sw, dw, sem):
            e0 = s * _EW + w * _WIN
            pltpu.async_copy(src_hbm.at[pl.ds(e0, _WIN)], sw, sem)
            pltpu.async_copy(dst_hbm.at[pl.ds(e0, _WIN)], dw, sem)

        def _drain(sw, dw, sem):
            pltpu.make_async_copy(src_hbm.at[pl.ds(0, _WIN)], sw, sem).wait()
            pltpu.make_async_copy(src_hbm.at[pl.ds(0, _WIN)], dw, sem).wait()

        _issue(0, srcw0, dstw0, esemA)

        def _pair(w2, carry):
            w = 2 * w2
            _issue(w + 1, srcw1, dstw1, esemB)
            _drain(srcw0, dstw0, esemA)
            carry = _process(srcw0, dstw0, carry)
            _issue(w + 2, srcw0, dstw0, esemA)
            _drain(srcw1, dstw1, esemB)
            carry = _process(srcw1, dstw1, carry)
            return carry

        zero = jnp.int32(0)
        carry = lax.fori_loop(0, (_NWIN - 1) // 2, _pair, (zero, zero, zero))
        _drain(srcw0, dstw0, esemA)
        cnt, p0, p1 = _process(srcw0, dstw0, carry)

        @pl.when(p0 > 0)
        def _():
            _drain_scatter(0)

        @pl.when(p1 > 0)
        def _():
            _drain_scatter(1)

        for k in range(_KB // 16):
            sel_src[pl.ds(cnt + k * 16, 16)] = pad_src
            sel_loc[pl.ds(cnt + k * 16, 16)] = pad_loc

        @pl.when(cnt > 0)
        def _():
            _stage_gather(0, 0)
            _wait_gather(0)
            _issue_scatter(0)
            _drain_scatter(0)

        plsc.subcore_barrier()
        rb = _RPP // _NSUB
        pltpu.sync_copy(acc.at[pl.ds(s * rb, rb)],
                        out_hbm.at[pl.ds(base + s * rb, rb)])
        return 0

    lax.fori_loop(0, _NPASS, _pass, 0)


@functools.partial(jax.jit, static_argnums=())
def _sc_msgdeg(src, dst, hp16):
    mesh = plsc.VectorSubcoreMesh(core_axis_name="c", subcore_axis_name="s")
    f = pl.kernel(
        _sc_body,
        out_type=jax.ShapeDtypeStruct((_NP, _MD16), _BF16),
        mesh=mesh,
        compiler_params=pltpu.CompilerParams(needs_layout_passes=False,
                                             use_tc_tiling_on_sc=False),
        scratch_types=[
            pltpu.VMEM((_WIN,), _I32),
            pltpu.VMEM((_WIN,), _I32),
            pltpu.VMEM((_WIN,), _I32),
            pltpu.VMEM((_WIN,), _I32),
            pltpu.VMEM((_SEL,), _I32),
            pltpu.VMEM((_SEL,), _I32),
            pltpu.VMEM((_KB,), _I32),
            pltpu.VMEM((_KB,), _I32),
            pltpu.VMEM((_KB,), _I32),
            pltpu.VMEM((_KB,), _I32),
            pltpu.VMEM((_KB, _MD16), _BF16),
            pltpu.VMEM((_KB, _MD16), _BF16),
            pltpu.VMEM((_ZR, _MD16), _BF16),
            pltpu.VMEM_SHARED((_SROWS, _MD16), _BF16),
            pltpu.SemaphoreType.DMA,
            pltpu.SemaphoreType.DMA,
            pltpu.SemaphoreType.DMA,
            pltpu.SemaphoreType.DMA,
            pltpu.SemaphoreType.DMA,
            pltpu.SemaphoreType.DMA,
            pltpu.SemaphoreType.DMA,
        ],
    )
    return f(src, dst, hp16)


def kernel(x, edge_index, batch, W1, b1, W2, b2, Wp, bp, Wc, bc):
    xp = jnp.zeros((_NP, 16), _F32).at[:_N, :11].set(x)
    W1p = jnp.zeros((16, _HID), _F32).at[:11, :].set(W1)
    hp, hp16 = _node_mlp1(xp, W1p, b1.reshape(1, _HID))

    mdp = _sc_msgdeg(edge_index[0], edge_index[1], hp16)

    btp = jnp.full((_NP,), _B, _I32).at[:_N].set(batch).reshape(_G, 1, _RB)
    sums, cnt = _node_mlp2_pool(hp, mdp, btp, W2, b2.reshape(1, _HID))
    logits = _head(sums, cnt, Wp, bp.reshape(1, _PROJ), Wc, bc.reshape(1, _NCLS))
    return jnp.broadcast_to(logits[None], (_SEQ, _B, _NCLS))


# speedup vs baseline: 7.1872x; 1.1701x over previous
import functools

import jax
import jax.numpy as jnp
from jax import lax
from jax.experimental import pallas as pl
from jax.experimental.pallas import tpu as pltpu
from jax.experimental.pallas import tpu_sc as plsc

_N = 50000
_E = 800000
_B = 64
_SEQ = 128
_HID = 256
_PROJ = 512
_NCLS = 1000

_RB = 1024
_NP = 51200
_G = _NP // _RB
_MD = 272
_MD16 = 288

_NCORE = 2
_NSUB = 16
_RPP = _NP // 10
_NPASS = 5
_SROWS = _RPP
_EW = _E // _NSUB
_WIN = 2000
_NWIN = _EW // _WIN
_KB = 128
_NBUF = 2
_SEL = 2176
_ZSH = _SROWS // _NSUB
_ZR = 8

_F32 = jnp.float32
_BF16 = jnp.bfloat16
_I32 = jnp.int32


def _mlp1_body(x_ref, w_ref, b_ref, o_ref, o16_ref):
    i = pl.program_id(0)
    h = jnp.dot(x_ref[...], w_ref[...], preferred_element_type=_F32)
    live = (i * _RB + lax.broadcasted_iota(_I32, (_RB, 1), 0)) < _N
    hr = jnp.where(live, jnp.maximum(h + b_ref[...], 0.0), 0.0)
    o_ref[:, :_HID] = hr
    lane = lax.broadcasted_iota(_I32, (_RB, _MD - _HID), 1)
    ones = jnp.where(live & (lane == 0), 1.0, 0.0).astype(_F32)
    o_ref[:, _HID:] = ones
    o16_ref[:, :_HID] = hr.astype(_BF16)
    lane16 = lax.broadcasted_iota(_I32, (_RB, _MD16 - _HID), 1)
    o16_ref[:, _HID:] = jnp.where(live & (lane16 == 0), 1.0, 0.0).astype(_BF16)


def _mlp2_pool_body(hp_ref, md_ref, bt_ref, w_ref, b_ref, sums_ref, cnt_ref):
    i = pl.program_id(0)
    msg = md_ref[:, :_HID].astype(_F32)
    deg = md_ref[:, _HID:_HID + 1].astype(_F32)
    m = msg / jnp.maximum(deg, 1.0)
    Hb = jnp.dot(hp_ref[:, :_HID] + m, w_ref[...], preferred_element_type=_F32)
    Hb = jnp.maximum(Hb + b_ref[...], 0.0)
    bt = bt_ref[0]
    seg = lax.broadcasted_iota(_I32, (_B, _RB), 0)
    mask = (seg == bt).astype(_F32)
    psum = jnp.dot(mask, Hb, preferred_element_type=_F32)
    pcnt = jnp.sum(mask, axis=1, keepdims=True)

    @pl.when(i == 0)
    def _init():
        sums_ref[...] = psum
        cnt_ref[...] = jnp.broadcast_to(pcnt, (_B, 128))

    @pl.when(i > 0)
    def _acc():
        sums_ref[...] += psum
        cnt_ref[...] += jnp.broadcast_to(pcnt, (_B, 128))


def _head_body(sums_ref, cnt_ref, wp_ref, bp_ref, wc_ref, bc_ref, o_ref):
    cnt = cnt_ref[:, 0:1]
    hag = sums_ref[...] / jnp.maximum(cnt, 1.0)
    t = jnp.dot(hag, wp_ref[...], preferred_element_type=_F32) + bp_ref[...]
    o_ref[...] = jnp.dot(t, wc_ref[...], preferred_element_type=_F32) + bc_ref[...]


def _node_mlp1(xp, W1p, b1):
    return pl.pallas_call(
        _mlp1_body,
        grid=(_G,),
        in_specs=[
            pl.BlockSpec((_RB, 16), lambda i: (i, 0)),
            pl.BlockSpec((16, _HID), lambda i: (0, 0)),
            pl.BlockSpec((1, _HID), lambda i: (0, 0)),
        ],
        out_specs=[
            pl.BlockSpec((_RB, _MD), lambda i: (i, 0)),
            pl.BlockSpec((_RB, _MD16), lambda i: (i, 0)),
        ],
        out_shape=[
            jax.ShapeDtypeStruct((_NP, _MD), _F32),
            jax.ShapeDtypeStruct((_NP, _MD16), _BF16),
        ],
    )(xp, W1p, b1)


def _node_mlp2_pool(hp, mdp, bt3, W2, b2):
    return pl.pallas_call(
        _mlp2_pool_body,
        grid=(_G,),
        in_specs=[
            pl.BlockSpec((_RB, _MD), lambda i: (i, 0)),
            pl.BlockSpec((_RB, _MD16), lambda i: (i, 0)),
            pl.BlockSpec((1, 1, _RB), lambda i: (i, 0, 0)),
            pl.BlockSpec((_HID, _HID), lambda i: (0, 0)),
            pl.BlockSpec((1, _HID), lambda i: (0, 0)),
        ],
        out_specs=[
            pl.BlockSpec((_B, _HID), lambda i: (0, 0)),
            pl.BlockSpec((_B, 128), lambda i: (0, 0)),
        ],
        out_shape=[
            jax.ShapeDtypeStruct((_B, _HID), _F32),
            jax.ShapeDtypeStruct((_B, 128), _F32),
        ],
    )(hp, mdp, bt3, W2, b2)


def _head(sums, cnt, Wp, bp, Wc, bc):
    return pl.pallas_call(
        _head_body,
        in_specs=[
            pl.BlockSpec((_B, _HID), lambda: (0, 0)),
            pl.BlockSpec((_B, 128), lambda: (0, 0)),
            pl.BlockSpec((_HID, _PROJ), lambda: (0, 0)),
            pl.BlockSpec((1, _PROJ), lambda: (0, 0)),
            pl.BlockSpec((_PROJ, _NCLS), lambda: (0, 0)),
            pl.BlockSpec((1, _NCLS), lambda: (0, 0)),
        ],
        out_specs=pl.BlockSpec((_B, _NCLS), lambda: (0, 0)),
        out_shape=jax.ShapeDtypeStruct((_B, _NCLS), _F32),
    )(sums, cnt, Wp, bp, Wc, bc)


def _sc_body(src_hbm, dst_hbm, hp_hbm, out_hbm,
             srcw0, dstw0, srcw1, dstw1, sel_src, sel_loc,
             srcb0, locb0, srcb1, locb1, rows0, rows1, zbuf,
             acc, esemA, esemB, gsem0, gsem1, ssem0, ssem1, zsem):
    c = lax.axis_index("c")
    s = lax.axis_index("s")
    lanes = lax.broadcasted_iota(_I32, (16,), 0)
    pad_src = _N + lanes * 8
    pad_loc = lanes
    srcbs, locbs, rowss = (srcb0, srcb1), (locb0, locb1), (rows0, rows1)
    gsems, ssems = (gsem0, gsem1), (ssem0, ssem1)

    def _zb(i, _):
        r = i // (_MD16 // 32)
        k = i - r * (_MD16 // 32)
        zbuf[r, pl.ds(k * 32, 32)] = jnp.zeros((32,), _BF16)
        return 0
    lax.fori_loop(0, _ZR * (_MD16 // 32), _zb, 0)

    def _stage_gather(t, j):
        for k in range(_KB // 16):
            srcbs[t][pl.ds(k * 16, 16)] = sel_src[pl.ds(j * _KB + k * 16, 16)]
            locbs[t][pl.ds(k * 16, 16)] = sel_loc[pl.ds(j * _KB + k * 16, 16)]
        pltpu.async_copy(hp_hbm.at[srcbs[t]], rowss[t], gsems[t])

    def _wait_gather(t):
        pltpu.make_async_copy(hp_hbm.at[srcbs[t]], rowss[t], gsems[t]).wait()

    def _issue_scatter(t):
        pltpu.async_copy(rowss[t], acc.at[locbs[t]], ssems[t], add=True)

    def _drain_scatter(t):
        pltpu.make_async_copy(rowss[t], acc.at[locbs[t]], ssems[t]).wait()

    def _pass(p, _):
        base = (2 * p + c) * _RPP

        plsc.subcore_barrier()
        z0 = s * _ZSH
        zds = [pltpu.async_copy(zbuf, acc.at[pl.ds(z0 + _ZR * k, _ZR)], zsem)
               for k in range(_ZSH // _ZR)]
        for d in zds:
            d.wait()
        plsc.subcore_barrier()

        def _process(sw, dw, carry):
            cnt, p0, p1 = carry
            pends = (p0, p1)

            def _compact(i, cv):
                d = dw[pl.ds(i * 16, 16)]
                sv = sw[pl.ds(i * 16, 16)]
                loc = d - base
                m = (loc >= 0) & (loc < _RPP)
                mi = jnp.where(m, jnp.int32(1), jnp.int32(0))
                pos = plsc.cumsum(mi) - mi + cv
                plsc.store_scatter(sel_loc, [pos], loc, mask=m)
                plsc.store_scatter(sel_src, [pos], sv, mask=m)
                return cv + plsc.all_reduce_population_count(m)
            cnt_v = lax.fori_loop(0, _WIN // 16, _compact,
                                  jnp.broadcast_to(cnt, (16,)))
            cnt = jnp.max(cnt_v)

            nf = cnt // _KB

            def _grp(g, _):
                for t in range(_NBUF):
                    j = _NBUF * g + t
                    @pl.when(j < nf)
                    def _():
                        @pl.when((g > 0) | (pends[t] > 0))
                        def _():
                            _drain_scatter(t)
                        _stage_gather(t, j)
                for t in range(_NBUF):
                    j = _NBUF * g + t
                    @pl.when(j < nf)
                    def _():
                        _wait_gather(t)
                        _issue_scatter(t)
                return 0
            lax.fori_loop(0, (nf + _NBUF - 1) // _NBUF, _grp, 0)

            roff = nf * _KB
            for k in range(_KB // 16):
                sv = sel_src[pl.ds(roff + k * 16, 16)]
                lv = sel_loc[pl.ds(roff + k * 16, 16)]
                sel_src[pl.ds(k * 16, 16)] = sv
                sel_loc[pl.ds(k * 16, 16)] = lv
            p0 = jnp.where(nf >= 1, jnp.int32(1), p0)
            p1 = jnp.where(nf >= 2, jnp.int32(1), p1)
            return cnt - roff, p0, p1

        def _issue(w, sw, dw, sem):
            e0 = s * _EW + w * _WIN
            pltpu.async_copy(src_hbm.at[pl.ds(e0, _WIN)], sw, sem)
            pltpu.async_copy(dst_hbm.at[pl.ds(e0, _WIN)], dw, sem)

        def _drain(sw, dw, sem):
            pltpu.make_async_copy(src_hbm.at[pl.ds(0, _WIN)], sw, sem).wait()
            pltpu.make_async_copy(src_hbm.at[pl.ds(0, _WIN)], dw, sem).wait()

        _issue(0, srcw0, dstw0, esemA)

        def _pair(w2, carry):
            w = 2 * w2
            _issue(w + 1, srcw1, dstw1, esemB)
            _drain(srcw0, dstw0, esemA)
            carry = _process(srcw0, dstw0, carry)
            _issue(w + 2, srcw0, dstw0, esemA)
            _drain(srcw1, dstw1, esemB)
            carry = _process(srcw1, dstw1, carry)
            return carry

        zero = jnp.int32(0)
        carry = lax.fori_loop(0, (_NWIN - 1) // 2, _pair, (zero, zero, zero))
        _drain(srcw0, dstw0, esemA)
        cnt, p0, p1 = _process(srcw0, dstw0, carry)

        @pl.when(p0 > 0)
        def _():
            _drain_scatter(0)

        @pl.when(p1 > 0)
        def _():
            _drain_scatter(1)

        for k in range(_KB // 16):
            sel_src[pl.ds(cnt + k * 16, 16)] = pad_src
            sel_loc[pl.ds(cnt + k * 16, 16)] = pad_loc

        @pl.when(cnt > 0)
        def _():
            _stage_gather(0, 0)
            _wait_gather(0)
            _issue_scatter(0)
            _drain_scatter(0)

        plsc.subcore_barrier()
        rb = _RPP // _NSUB
        pltpu.sync_copy(acc.at[pl.ds(s * rb, rb)],
                        out_hbm.at[pl.ds(base + s * rb, rb)])
        return 0

    lax.fori_loop(0, _NPASS, _pass, 0)


@functools.partial(jax.jit, static_argnums=())
def _sc_msgdeg(src, dst, hp16):
    mesh = plsc.VectorSubcoreMesh(core_axis_name="c", subcore_axis_name="s")
    f = pl.kernel(
        _sc_body,
        out_type=jax.ShapeDtypeStruct((_NP, _MD16), _BF16),
        mesh=mesh,
        compiler_params=pltpu.CompilerParams(needs_layout_passes=False,
                                             use_tc_tiling_on_sc=False),
        scratch_types=[
            pltpu.VMEM((_WIN,), _I32),
            pltpu.VMEM((_WIN,), _I32),
            pltpu.VMEM((_WIN,), _I32),
            pltpu.VMEM((_WIN,), _I32),
            pltpu.VMEM((_SEL,), _I32),
            pltpu.VMEM((_SEL,), _I32),
            pltpu.VMEM((_KB,), _I32),
            pltpu.VMEM((_KB,), _I32),
            pltpu.VMEM((_KB,), _I32),
            pltpu.VMEM((_KB,), _I32),
            pltpu.VMEM((_KB, _MD16), _BF16),
            pltpu.VMEM((_KB, _MD16), _BF16),
            pltpu.VMEM((_ZR, _MD16), _BF16),
            pltpu.VMEM_SHARED((_SROWS, _MD16), _BF16),
            pltpu.SemaphoreType.DMA,
            pltpu.SemaphoreType.DMA,
            pltpu.SemaphoreType.DMA,
            pltpu.SemaphoreType.DMA,
            pltpu.SemaphoreType.DMA,
            pltpu.SemaphoreType.DMA,
            pltpu.SemaphoreType.DMA,
        ],
    )
    return f(src, dst, hp16)


def kernel(x, edge_index, batch, W1, b1, W2, b2, Wp, bp, Wc, bc):
    xp = jnp.zeros((_NP, 16), _F32).at[:_N, :11].set(x)
    W1p = jnp.zeros((16, _HID), _F32).at[:11, :].set(W1)
    hp, hp16 = _node_mlp1(xp, W1p, b1.reshape(1, _HID))

    mdp = _sc_msgdeg(edge_index[0], edge_index[1], hp16)

    btp = jnp.full((_NP,), _B, _I32).at[:_N].set(batch).reshape(_G, 1, _RB)
    sums, cnt = _node_mlp2_pool(hp, mdp, btp, W2, b2.reshape(1, _HID))
    logits = _head(sums, cnt, Wp, bp.reshape(1, _PROJ), Wc, bc.reshape(1, _NCLS))
    return jnp.broadcast_to(logits[None], (_SEQ, _B, _NCLS))


# speedup vs baseline: 7.1887x; 1.0002x over previous
import functools

import jax
import jax.numpy as jnp
from jax import lax
from jax.experimental import pallas as pl
from jax.experimental.pallas import tpu as pltpu
from jax.experimental.pallas import tpu_sc as plsc

_N = 50000
_E = 800000
_B = 64
_SEQ = 128
_HID = 256
_PROJ = 512
_NCLS = 1000

_RB = 1024
_NP = 51200
_G = _NP // _RB
_MD = 272
_MD16 = 288

_NCORE = 2
_NSUB = 16
_RPP = _NP // 10
_NPASS = 5
_SROWS = _RPP
_EW = _E // _NSUB
_WIN = 2000
_NWIN = _EW // _WIN
_KB = 128
_NBUF = 2
_SEL = 2176
_ZSH = _SROWS // _NSUB
_ZR = 8

_F32 = jnp.float32
_BF16 = jnp.bfloat16
_I32 = jnp.int32


def _mlp1_body(x_ref, w_ref, b_ref, o_ref, o16_ref):
    i = pl.program_id(0)
    h = jnp.dot(x_ref[...], w_ref[...], preferred_element_type=_F32)
    live = (i * _RB + lax.broadcasted_iota(_I32, (_RB, 1), 0)) < _N
    hr = jnp.where(live, jnp.maximum(h + b_ref[...], 0.0), 0.0)
    o_ref[:, :_HID] = hr
    lane = lax.broadcasted_iota(_I32, (_RB, _MD - _HID), 1)
    ones = jnp.where(live & (lane == 0), 1.0, 0.0).astype(_F32)
    o_ref[:, _HID:] = ones
    o16_ref[:, :_HID] = hr.astype(_BF16)
    lane16 = lax.broadcasted_iota(_I32, (_RB, _MD16 - _HID), 1)
    o16_ref[:, _HID:] = jnp.where(live & (lane16 == 0), 1.0, 0.0).astype(_BF16)


def _mlp2_pool_body(hp_ref, md_ref, bt_ref, w_ref, b_ref, sums_ref, cnt_ref):
    i = pl.program_id(0)
    msg = md_ref[:, :_HID].astype(_F32)
    deg = md_ref[:, _HID:_HID + 1].astype(_F32)
    m = msg / jnp.maximum(deg, 1.0)
    Hb = jnp.dot((hp_ref[:, :_HID] + m).astype(_BF16),
                 w_ref[...].astype(_BF16), preferred_element_type=_F32)
    Hb = jnp.maximum(Hb + b_ref[...], 0.0)
    bt = bt_ref[0]
    seg = lax.broadcasted_iota(_I32, (_B, _RB), 0)
    mask = (seg == bt).astype(_BF16)
    psum = jnp.dot(mask, Hb.astype(_BF16), preferred_element_type=_F32)
    pcnt = jnp.sum(mask.astype(_F32), axis=1, keepdims=True)

    @pl.when(i == 0)
    def _init():
        sums_ref[...] = psum
        cnt_ref[...] = jnp.broadcast_to(pcnt, (_B, 128))

    @pl.when(i > 0)
    def _acc():
        sums_ref[...] += psum
        cnt_ref[...] += jnp.broadcast_to(pcnt, (_B, 128))


def _head_body(sums_ref, cnt_ref, wp_ref, bp_ref, wc_ref, bc_ref, o_ref):
    cnt = cnt_ref[:, 0:1]
    hag = sums_ref[...] / jnp.maximum(cnt, 1.0)
    t = jnp.dot(hag, wp_ref[...], preferred_element_type=_F32) + bp_ref[...]
    o_ref[...] = jnp.dot(t, wc_ref[...], preferred_element_type=_F32) + bc_ref[...]


def _node_mlp1(xp, W1p, b1):
    return pl.pallas_call(
        _mlp1_body,
        grid=(_G,),
        in_specs=[
            pl.BlockSpec((_RB, 16), lambda i: (i, 0)),
            pl.BlockSpec((16, _HID), lambda i: (0, 0)),
            pl.BlockSpec((1, _HID), lambda i: (0, 0)),
        ],
        out_specs=[
            pl.BlockSpec((_RB, _MD), lambda i: (i, 0)),
            pl.BlockSpec((_RB, _MD16), lambda i: (i, 0)),
        ],
        out_shape=[
            jax.ShapeDtypeStruct((_NP, _MD), _F32),
            jax.ShapeDtypeStruct((_NP, _MD16), _BF16),
        ],
    )(xp, W1p, b1)


def _node_mlp2_pool(hp, mdp, bt3, W2, b2):
    return pl.pallas_call(
        _mlp2_pool_body,
        grid=(_G,),
        in_specs=[
            pl.BlockSpec((_RB, _MD), lambda i: (i, 0)),
            pl.BlockSpec((_RB, _MD16), lambda i: (i, 0)),
            pl.BlockSpec((1, 1, _RB), lambda i: (i, 0, 0)),
            pl.BlockSpec((_HID, _HID), lambda i: (0, 0)),
            pl.BlockSpec((1, _HID), lambda i: (0, 0)),
        ],
        out_specs=[
            pl.BlockSpec((_B, _HID), lambda i: (0, 0)),
            pl.BlockSpec((_B, 128), lambda i: (0, 0)),
        ],
        out_shape=[
            jax.ShapeDtypeStruct((_B, _HID), _F32),
            jax.ShapeDtypeStruct((_B, 128), _F32),
        ],
    )(hp, mdp, bt3, W2, b2)


def _head(sums, cnt, Wp, bp, Wc, bc):
    return pl.pallas_call(
        _head_body,
        in_specs=[
            pl.BlockSpec((_B, _HID), lambda: (0, 0)),
            pl.BlockSpec((_B, 128), lambda: (0, 0)),
            pl.BlockSpec((_HID, _PROJ), lambda: (0, 0)),
            pl.BlockSpec((1, _PROJ), lambda: (0, 0)),
            pl.BlockSpec((_PROJ, _NCLS), lambda: (0, 0)),
            pl.BlockSpec((1, _NCLS), lambda: (0, 0)),
        ],
        out_specs=pl.BlockSpec((_B, _NCLS), lambda: (0, 0)),
        out_shape=jax.ShapeDtypeStruct((_B, _NCLS), _F32),
    )(sums, cnt, Wp, bp, Wc, bc)


def _sc_body(src_hbm, dst_hbm, hp_hbm, out_hbm,
             srcw0, dstw0, srcw1, dstw1, sel_src, sel_loc,
             srcb0, locb0, srcb1, locb1, rows0, rows1, zbuf,
             acc, esemA, esemB, gsem0, gsem1, ssem0, ssem1, zsem):
    c = lax.axis_index("c")
    s = lax.axis_index("s")
    lanes = lax.broadcasted_iota(_I32, (16,), 0)
    pad_src = _N + lanes * 8
    pad_loc = lanes
    srcbs, locbs, rowss = (srcb0, srcb1), (locb0, locb1), (rows0, rows1)
    gsems, ssems = (gsem0, gsem1), (ssem0, ssem1)

    def _zb(i, _):
        r = i // (_MD16 // 32)
        k = i - r * (_MD16 // 32)
        zbuf[r, pl.ds(k * 32, 32)] = jnp.zeros((32,), _BF16)
        return 0
    lax.fori_loop(0, _ZR * (_MD16 // 32), _zb, 0)

    def _stage_gather(t, j):
        for k in range(_KB // 16):
            srcbs[t][pl.ds(k * 16, 16)] = sel_src[pl.ds(j * _KB + k * 16, 16)]
            locbs[t][pl.ds(k * 16, 16)] = sel_loc[pl.ds(j * _KB + k * 16, 16)]
        pltpu.async_copy(hp_hbm.at[srcbs[t]], rowss[t], gsems[t])

    def _wait_gather(t):
        pltpu.make_async_copy(hp_hbm.at[srcbs[t]], rowss[t], gsems[t]).wait()

    def _issue_scatter(t):
        pltpu.async_copy(rowss[t], acc.at[locbs[t]], ssems[t], add=True)

    def _drain_scatter(t):
        pltpu.make_async_copy(rowss[t], acc.at[locbs[t]], ssems[t]).wait()

    def _pass(p, _):
        base = (2 * p + c) * _RPP

        plsc.subcore_barrier()
        z0 = s * _ZSH
        zds = [pltpu.async_copy(zbuf, acc.at[pl.ds(z0 + _ZR * k, _ZR)], zsem)
               for k in range(_ZSH // _ZR)]
        for d in zds:
            d.wait()
        plsc.subcore_barrier()

        def _process(sw, dw, carry):
            cnt, p0, p1 = carry
            pends = (p0, p1)

            def _compact(i, cv):
                d = dw[pl.ds(i * 16, 16)]
                sv = sw[pl.ds(i * 16, 16)]
                loc = d - base
                m = (loc >= 0) & (loc < _RPP)
                mi = jnp.where(m, jnp.int32(1), jnp.int32(0))
                pos = plsc.cumsum(mi) - mi + cv
                plsc.store_scatter(sel_loc, [pos], loc, mask=m)
                plsc.store_scatter(sel_src, [pos], sv, mask=m)
                return cv + plsc.all_reduce_population_count(m)
            cnt_v = lax.fori_loop(0, _WIN // 16, _compact,
                                  jnp.broadcast_to(cnt, (16,)))
            cnt = jnp.max(cnt_v)

            nf = cnt // _KB

            def _grp(g, _):
                for t in range(_NBUF):
                    j = _NBUF * g + t
                    @pl.when(j < nf)
                    def _():
                        @pl.when((g > 0) | (pends[t] > 0))
                        def _():
                            _drain_scatter(t)
                        _stage_gather(t, j)
                for t in range(_NBUF):
                    j = _NBUF * g + t
                    @pl.when(j < nf)
                    def _():
                        _wait_gather(t)
                        _issue_scatter(t)
                return 0
            lax.fori_loop(0, (nf + _NBUF - 1) // _NBUF, _grp, 0)

            roff = nf * _KB
            for k in range(_KB // 16):
                sv = sel_src[pl.ds(roff + k * 16, 16)]
                lv = sel_loc[pl.ds(roff + k * 16, 16)]
                sel_src[pl.ds(k * 16, 16)] = sv
                sel_loc[pl.ds(k * 16, 16)] = lv
            p0 = jnp.where(nf >= 1, jnp.int32(1), p0)
            p1 = jnp.where(nf >= 2, jnp.int32(1), p1)
            return cnt - roff, p0, p1

        def _issue(w, sw, dw, sem):
            e0 = s * _EW + w * _WIN
            pltpu.async_copy(src_hbm.at[pl.ds(e0, _WIN)], sw, sem)
            pltpu.async_copy(dst_hbm.at[pl.ds(e0, _WIN)], dw, sem)

        def _drain(sw, dw, sem):
            pltpu.make_async_copy(src_hbm.at[pl.ds(0, _WIN)], sw, sem).wait()
            pltpu.make_async_copy(src_hbm.at[pl.ds(0, _WIN)], dw, sem).wait()

        _issue(0, srcw0, dstw0, esemA)

        def _pair(w2, carry):
            w = 2 * w2
            _issue(w + 1, srcw1, dstw1, esemB)
            _drain(srcw0, dstw0, esemA)
            carry = _process(srcw0, dstw0, carry)
            _issue(w + 2, srcw0, dstw0, esemA)
            _drain(srcw1, dstw1, esemB)
            carry = _process(srcw1, dstw1, carry)
            return carry

        zero = jnp.int32(0)
        carry = lax.fori_loop(0, (_NWIN - 1) // 2, _pair, (zero, zero, zero))
        _drain(srcw0, dstw0, esemA)
        cnt, p0, p1 = _process(srcw0, dstw0, carry)

        @pl.when(p0 > 0)
        def _():
            _drain_scatter(0)

        @pl.when(p1 > 0)
        def _():
            _drain_scatter(1)

        for k in range(_KB // 16):
            sel_src[pl.ds(cnt + k * 16, 16)] = pad_src
            sel_loc[pl.ds(cnt + k * 16, 16)] = pad_loc

        @pl.when(cnt > 0)
        def _():
            _stage_gather(0, 0)
            _wait_gather(0)
            _issue_scatter(0)
            _drain_scatter(0)

        plsc.subcore_barrier()
        rb = _RPP // _NSUB
        pltpu.sync_copy(acc.at[pl.ds(s * rb, rb)],
                        out_hbm.at[pl.ds(base + s * rb, rb)])
        return 0

    lax.fori_loop(0, _NPASS, _pass, 0)


@functools.partial(jax.jit, static_argnums=())
def _sc_msgdeg(src, dst, hp16):
    mesh = plsc.VectorSubcoreMesh(core_axis_name="c", subcore_axis_name="s")
    f = pl.kernel(
        _sc_body,
        out_type=jax.ShapeDtypeStruct((_NP, _MD16), _BF16),
        mesh=mesh,
        compiler_params=pltpu.CompilerParams(needs_layout_passes=False,
                                             use_tc_tiling_on_sc=False),
        scratch_types=[
            pltpu.VMEM((_WIN,), _I32),
            pltpu.VMEM((_WIN,), _I32),
            pltpu.VMEM((_WIN,), _I32),
            pltpu.VMEM((_WIN,), _I32),
            pltpu.VMEM((_SEL,), _I32),
            pltpu.VMEM((_SEL,), _I32),
            pltpu.VMEM((_KB,), _I32),
            pltpu.VMEM((_KB,), _I32),
            pltpu.VMEM((_KB,), _I32),
            pltpu.VMEM((_KB,), _I32),
            pltpu.VMEM((_KB, _MD16), _BF16),
            pltpu.VMEM((_KB, _MD16), _BF16),
            pltpu.VMEM((_ZR, _MD16), _BF16),
            pltpu.VMEM_SHARED((_SROWS, _MD16), _BF16),
            pltpu.SemaphoreType.DMA,
            pltpu.SemaphoreType.DMA,
            pltpu.SemaphoreType.DMA,
            pltpu.SemaphoreType.DMA,
            pltpu.SemaphoreType.DMA,
            pltpu.SemaphoreType.DMA,
            pltpu.SemaphoreType.DMA,
        ],
    )
    return f(src, dst, hp16)


def kernel(x, edge_index, batch, W1, b1, W2, b2, Wp, bp, Wc, bc):
    xp = jnp.zeros((_NP, 16), _F32).at[:_N, :11].set(x)
    W1p = jnp.zeros((16, _HID), _F32).at[:11, :].set(W1)
    hp, hp16 = _node_mlp1(xp, W1p, b1.reshape(1, _HID))

    mdp = _sc_msgdeg(edge_index[0], edge_index[1], hp16)

    btp = jnp.full((_NP,), _B, _I32).at[:_N].set(batch).reshape(_G, 1, _RB)
    sums, cnt = _node_mlp2_pool(hp, mdp, btp, W2, b2.reshape(1, _HID))
    logits = _head(sums, cnt, Wp, bp.reshape(1, _PROJ), Wc, bc.reshape(1, _NCLS))
    return jnp.broadcast_to(logits[None], (_SEQ, _B, _NCLS))


# speedup vs baseline: 7.4067x; 1.0303x over previous
import functools

import jax
import jax.numpy as jnp
from jax import lax
from jax.experimental import pallas as pl
from jax.experimental.pallas import tpu as pltpu
from jax.experimental.pallas import tpu_sc as plsc

_N = 50000
_E = 800000
_B = 64
_SEQ = 128
_HID = 256
_PROJ = 512
_NCLS = 1000

_RB = 1024
_NP = 51200
_G = _NP // _RB
_MD = 272
_MD16 = 288

_NCORE = 2
_NSUB = 16
_RPP = _NP // 10
_NPASS = 5
_SROWS = _RPP
_EW = _E // _NSUB
_WIN = 2000
_NWIN = _EW // _WIN
_KB = 128
_NBUF = 2
_SEL = 2176
_ZSH = _SROWS // _NSUB
_ZR = 8

_F32 = jnp.float32
_BF16 = jnp.bfloat16
_I32 = jnp.int32


def _mlp1_body(x_ref, w_ref, b_ref, o16_ref):
    i = pl.program_id(0)
    h = jnp.dot(x_ref[...], w_ref[...], preferred_element_type=_F32)
    live = (i * _RB + lax.broadcasted_iota(_I32, (_RB, 1), 0)) < _N
    hr = jnp.where(live, jnp.maximum(h + b_ref[...], 0.0), 0.0)
    o16_ref[:, :_HID] = hr.astype(_BF16)
    lane16 = lax.broadcasted_iota(_I32, (_RB, _MD16 - _HID), 1)
    o16_ref[:, _HID:] = jnp.where(live & (lane16 == 0), 1.0, 0.0).astype(_BF16)


def _mlp2_pool_body(x_ref, w1_ref, b1_ref, md_ref, bt_ref, w_ref, b_ref,
                    sums_ref, cnt_ref):
    i = pl.program_id(0)
    h = jnp.maximum(jnp.dot(x_ref[...], w1_ref[...],
                            preferred_element_type=_F32) + b1_ref[...], 0.0)
    msg = md_ref[:, :_HID].astype(_F32)
    deg = md_ref[:, _HID:_HID + 1].astype(_F32)
    m = msg / jnp.maximum(deg, 1.0)
    Hb = jnp.dot((h + m).astype(_BF16),
                 w_ref[...].astype(_BF16), preferred_element_type=_F32)
    Hb = jnp.maximum(Hb + b_ref[...], 0.0)
    bt = bt_ref[0]
    seg = lax.broadcasted_iota(_I32, (_B, _RB), 0)
    mask = (seg == bt).astype(_BF16)
    psum = jnp.dot(mask, Hb.astype(_BF16), preferred_element_type=_F32)
    pcnt = jnp.sum(mask.astype(_F32), axis=1, keepdims=True)

    @pl.when(i == 0)
    def _init():
        sums_ref[...] = psum
        cnt_ref[...] = jnp.broadcast_to(pcnt, (_B, 128))

    @pl.when(i > 0)
    def _acc():
        sums_ref[...] += psum
        cnt_ref[...] += jnp.broadcast_to(pcnt, (_B, 128))


def _head_body(sums_ref, cnt_ref, wp_ref, bp_ref, wc_ref, bc_ref, o_ref):
    cnt = cnt_ref[:, 0:1]
    hag = sums_ref[...] / jnp.maximum(cnt, 1.0)
    t = jnp.dot(hag, wp_ref[...], preferred_element_type=_F32) + bp_ref[...]
    o_ref[...] = jnp.dot(t, wc_ref[...], preferred_element_type=_F32) + bc_ref[...]


def _node_mlp1(xp, W1p, b1):
    return pl.pallas_call(
        _mlp1_body,
        grid=(_G,),
        in_specs=[
            pl.BlockSpec((_RB, 16), lambda i: (i, 0)),
            pl.BlockSpec((16, _HID), lambda i: (0, 0)),
            pl.BlockSpec((1, _HID), lambda i: (0, 0)),
        ],
        out_specs=pl.BlockSpec((_RB, _MD16), lambda i: (i, 0)),
        out_shape=jax.ShapeDtypeStruct((_NP, _MD16), _BF16),
    )(xp, W1p, b1)


def _node_mlp2_pool(xp, W1p, b1, mdp, bt3, W2, b2):
    return pl.pallas_call(
        _mlp2_pool_body,
        grid=(_G,),
        in_specs=[
            pl.BlockSpec((_RB, 16), lambda i: (i, 0)),
            pl.BlockSpec((16, _HID), lambda i: (0, 0)),
            pl.BlockSpec((1, _HID), lambda i: (0, 0)),
            pl.BlockSpec((_RB, _MD16), lambda i: (i, 0)),
            pl.BlockSpec((1, 1, _RB), lambda i: (i, 0, 0)),
            pl.BlockSpec((_HID, _HID), lambda i: (0, 0)),
            pl.BlockSpec((1, _HID), lambda i: (0, 0)),
        ],
        out_specs=[
            pl.BlockSpec((_B, _HID), lambda i: (0, 0)),
            pl.BlockSpec((_B, 128), lambda i: (0, 0)),
        ],
        out_shape=[
            jax.ShapeDtypeStruct((_B, _HID), _F32),
            jax.ShapeDtypeStruct((_B, 128), _F32),
        ],
    )(xp, W1p, b1, mdp, bt3, W2, b2)


def _head(sums, cnt, Wp, bp, Wc, bc):
    return pl.pallas_call(
        _head_body,
        in_specs=[
            pl.BlockSpec((_B, _HID), lambda: (0, 0)),
            pl.BlockSpec((_B, 128), lambda: (0, 0)),
            pl.BlockSpec((_HID, _PROJ), lambda: (0, 0)),
            pl.BlockSpec((1, _PROJ), lambda: (0, 0)),
            pl.BlockSpec((_PROJ, _NCLS), lambda: (0, 0)),
            pl.BlockSpec((1, _NCLS), lambda: (0, 0)),
        ],
        out_specs=pl.BlockSpec((_B, _NCLS), lambda: (0, 0)),
        out_shape=jax.ShapeDtypeStruct((_B, _NCLS), _F32),
    )(sums, cnt, Wp, bp, Wc, bc)


def _sc_body(src_hbm, dst_hbm, hp_hbm, out_hbm,
             srcw0, dstw0, srcw1, dstw1, sel_src, sel_loc,
             srcb0, locb0, srcb1, locb1, rows0, rows1, zbuf,
             acc, esemA, esemB, gsem0, gsem1, ssem0, ssem1, zsem):
    c = lax.axis_index("c")
    s = lax.axis_index("s")
    lanes = lax.broadcasted_iota(_I32, (16,), 0)
    pad_src = _N + lanes * 8
    pad_loc = lanes
    srcbs, locbs, rowss = (srcb0, srcb1), (locb0, locb1), (rows0, rows1)
    gsems, ssems = (gsem0, gsem1), (ssem0, ssem1)

    def _zb(i, _):
        r = i // (_MD16 // 32)
        k = i - r * (_MD16 // 32)
        zbuf[r, pl.ds(k * 32, 32)] = jnp.zeros((32,), _BF16)
        return 0
    lax.fori_loop(0, _ZR * (_MD16 // 32), _zb, 0)

    def _stage_gather(t, j):
        for k in range(_KB // 16):
            srcbs[t][pl.ds(k * 16, 16)] = sel_src[pl.ds(j * _KB + k * 16, 16)]
            locbs[t][pl.ds(k * 16, 16)] = sel_loc[pl.ds(j * _KB + k * 16, 16)]
        pltpu.async_copy(hp_hbm.at[srcbs[t]], rowss[t], gsems[t])

    def _wait_gather(t):
        pltpu.make_async_copy(hp_hbm.at[srcbs[t]], rowss[t], gsems[t]).wait()

    def _issue_scatter(t):
        pltpu.async_copy(rowss[t], acc.at[locbs[t]], ssems[t], add=True)

    def _drain_scatter(t):
        pltpu.make_async_copy(rowss[t], acc.at[locbs[t]], ssems[t]).wait()

    def _pass(p, _):
        base = (2 * p + c) * _RPP

        plsc.subcore_barrier()
        z0 = s * _ZSH
        zds = [pltpu.async_copy(zbuf, acc.at[pl.ds(z0 + _ZR * k, _ZR)], zsem)
               for k in range(_ZSH // _ZR)]
        for d in zds:
            d.wait()
        plsc.subcore_barrier()

        def _process(sw, dw, carry):
            cnt, p0, p1 = carry
            pends = (p0, p1)

            def _compact(i, cv):
                d = dw[pl.ds(i * 16, 16)]
                sv = sw[pl.ds(i * 16, 16)]
                loc = d - base
                m = (loc >= 0) & (loc < _RPP)
                mi = jnp.where(m, jnp.int32(1), jnp.int32(0))
                pos = plsc.cumsum(mi) - mi + cv
                plsc.store_scatter(sel_loc, [pos], loc, mask=m)
                plsc.store_scatter(sel_src, [pos], sv, mask=m)
                return cv + plsc.all_reduce_population_count(m)
            cnt_v = lax.fori_loop(0, _WIN // 16, _compact,
                                  jnp.broadcast_to(cnt, (16,)))
            cnt = jnp.max(cnt_v)

            nf = cnt // _KB

            def _grp(g, _):
                for t in range(_NBUF):
                    j = _NBUF * g + t
                    @pl.when(j < nf)
                    def _():
                        @pl.when((g > 0) | (pends[t] > 0))
                        def _():
                            _drain_scatter(t)
                        _stage_gather(t, j)
                for t in range(_NBUF):
                    j = _NBUF * g + t
                    @pl.when(j < nf)
                    def _():
                        _wait_gather(t)
                        _issue_scatter(t)
                return 0
            lax.fori_loop(0, (nf + _NBUF - 1) // _NBUF, _grp, 0)

            roff = nf * _KB
            for k in range(_KB // 16):
                sv = sel_src[pl.ds(roff + k * 16, 16)]
                lv = sel_loc[pl.ds(roff + k * 16, 16)]
                sel_src[pl.ds(k * 16, 16)] = sv
                sel_loc[pl.ds(k * 16, 16)] = lv
            p0 = jnp.where(nf >= 1, jnp.int32(1), p0)
            p1 = jnp.where(nf >= 2, jnp.int32(1), p1)
            return cnt - roff, p0, p1

        def _issue(w, sw, dw, sem):
            e0 = s * _EW + w * _WIN
            pltpu.async_copy(src_hbm.at[pl.ds(e0, _WIN)], sw, sem)
            pltpu.async_copy(dst_hbm.at[pl.ds(e0, _WIN)], dw, sem)

        def _drain(sw, dw, sem):
            pltpu.make_async_copy(src_hbm.at[pl.ds(0, _WIN)], sw, sem).wait()
            pltpu.make_async_copy(src_hbm.at[pl.ds(0, _WIN)], dw, sem).wait()

        _issue(0, srcw0, dstw0, esemA)

        def _pair(w2, carry):
            w = 2 * w2
            _issue(w + 1, srcw1, dstw1, esemB)
            _drain(srcw0, dstw0, esemA)
            carry = _process(srcw0, dstw0, carry)
            _issue(w + 2, srcw0, dstw0, esemA)
            _drain(srcw1, dstw1, esemB)
            carry = _process(srcw1, dstw1, carry)
            return carry

        zero = jnp.int32(0)
        carry = lax.fori_loop(0, (_NWIN - 1) // 2, _pair, (zero, zero, zero))
        _drain(srcw0, dstw0, esemA)
        cnt, p0, p1 = _process(srcw0, dstw0, carry)

        @pl.when(p0 > 0)
        def _():
            _drain_scatter(0)

        @pl.when(p1 > 0)
        def _():
            _drain_scatter(1)

        for k in range(_KB // 16):
            sel_src[pl.ds(cnt + k * 16, 16)] = pad_src
            sel_loc[pl.ds(cnt + k * 16, 16)] = pad_loc

        @pl.when(cnt > 0)
        def _():
            _stage_gather(0, 0)
            _wait_gather(0)
            _issue_scatter(0)
            _drain_scatter(0)

        plsc.subcore_barrier()
        rb = _RPP // _NSUB
        pltpu.sync_copy(acc.at[pl.ds(s * rb, rb)],
                        out_hbm.at[pl.ds(base + s * rb, rb)])
        return 0

    lax.fori_loop(0, _NPASS, _pass, 0)


@functools.partial(jax.jit, static_argnums=())
def _sc_msgdeg(src, dst, hp16):
    mesh = plsc.VectorSubcoreMesh(core_axis_name="c", subcore_axis_name="s")
    f = pl.kernel(
        _sc_body,
        out_type=jax.ShapeDtypeStruct((_NP, _MD16), _BF16),
        mesh=mesh,
        compiler_params=pltpu.CompilerParams(needs_layout_passes=False,
                                             use_tc_tiling_on_sc=False),
        scratch_types=[
            pltpu.VMEM((_WIN,), _I32),
            pltpu.VMEM((_WIN,), _I32),
            pltpu.VMEM((_WIN,), _I32),
            pltpu.VMEM((_WIN,), _I32),
            pltpu.VMEM((_SEL,), _I32),
            pltpu.VMEM((_SEL,), _I32),
            pltpu.VMEM((_KB,), _I32),
            pltpu.VMEM((_KB,), _I32),
            pltpu.VMEM((_KB,), _I32),
            pltpu.VMEM((_KB,), _I32),
            pltpu.VMEM((_KB, _MD16), _BF16),
            pltpu.VMEM((_KB, _MD16), _BF16),
            pltpu.VMEM((_ZR, _MD16), _BF16),
            pltpu.VMEM_SHARED((_SROWS, _MD16), _BF16),
            pltpu.SemaphoreType.DMA,
            pltpu.SemaphoreType.DMA,
            pltpu.SemaphoreType.DMA,
            pltpu.SemaphoreType.DMA,
            pltpu.SemaphoreType.DMA,
            pltpu.SemaphoreType.DMA,
            pltpu.SemaphoreType.DMA,
        ],
    )
    return f(src, dst, hp16)


def kernel(x, edge_index, batch, W1, b1, W2, b2, Wp, bp, Wc, bc):
    xp = jnp.zeros((_NP, 16), _F32).at[:_N, :11].set(x)
    W1p = jnp.zeros((16, _HID), _F32).at[:11, :].set(W1)
    hp16 = _node_mlp1(xp, W1p, b1.reshape(1, _HID))

    mdp = _sc_msgdeg(edge_index[0], edge_index[1], hp16)

    btp = jnp.full((_NP,), _B, _I32).at[:_N].set(batch).reshape(_G, 1, _RB)
    sums, cnt = _node_mlp2_pool(xp, W1p, b1.reshape(1, _HID), mdp, btp,
                                W2, b2.reshape(1, _HID))
    logits = _head(sums, cnt, Wp, bp.reshape(1, _PROJ), Wc, bc.reshape(1, _NCLS))
    return jnp.broadcast_to(logits[None], (_SEQ, _B, _NCLS))
